# Initial kernel scaffold; baseline (speedup 1.0000x reference)
import jax
import jax.numpy as jnp
from jax.experimental import pallas as pl


def kernel(x, edge_index, edge_attr, enc_n_W1, enc_n_b1, enc_n_W2, enc_n_b2, enc_n_g, enc_n_be, enc_e_W1, enc_e_b1, enc_e_W2, enc_e_b2, enc_e_g, enc_e_be, pe_W1, pe_b1, pe_W2, pe_b2, pe_g, pe_be, pn_W1, pn_b1, pn_W2, pn_b2, pn_g, pn_be, dec_W1, dec_b1, dec_W2, dec_b2):
    raise NotImplementedError("write your pallas kernel here")


# speedup vs baseline: 2.4895x; 2.4895x over previous
import functools

import jax
import jax.numpy as jnp
from jax import lax
from jax.experimental import pallas as pl
from jax.experimental.pallas import tpu as pltpu
from jax.experimental.pallas import tpu_sc as plsc

_N = 10000
_E = 320000
_H = 128
_NC = 2
_NS = 16
_NW = _NC * _NS
_CHUNK = 80

_F32 = jnp.float32


def _ln(u, g, b):
  mu = jnp.mean(u, axis=-1, keepdims=True)
  d = u - mu
  var = jnp.mean(d * d, axis=-1, keepdims=True)
  return d / jnp.sqrt(var + 1e-5) * g + b


def _dot(a, b):
  return jnp.dot(a, b, preferred_element_type=_F32)


def _mlp_ln_body(x_ref, w1_ref, b1_ref, w2_ref, b2_ref, g_ref, be_ref, o_ref):
  t = jnp.maximum(_dot(x_ref[...], w1_ref[...]) + b1_ref[...], 0.0)
  u = _dot(t, w2_ref[...]) + b2_ref[...]
  o_ref[...] = _ln(u, g_ref[...], be_ref[...])


def _mlp_ln(x, w1, b1, w2, b2, g, be, block):
  n, fin = x.shape
  h = w1.shape[1]
  full = lambda i: (0, 0)
  return pl.pallas_call(
      _mlp_ln_body,
      grid=(n // block,),
      in_specs=[
          pl.BlockSpec((block, fin), lambda i: (i, 0)),
          pl.BlockSpec((fin, h), full),
          pl.BlockSpec((1, h), full),
          pl.BlockSpec((h, h), full),
          pl.BlockSpec((1, h), full),
          pl.BlockSpec((1, h), full),
          pl.BlockSpec((1, h), full),
      ],
      out_specs=pl.BlockSpec((block, h), lambda i: (i, 0)),
      out_shape=jax.ShapeDtypeStruct((n, h), _F32),
  )(x, w1, b1.reshape(1, -1), w2, b2.reshape(1, -1), g.reshape(1, -1),
    be.reshape(1, -1))


def _ab_body(h_ref, ws_ref, wd_ref, o_ref):
  hh = h_ref[...]
  o_ref[0] = _dot(hh, ws_ref[...])
  o_ref[1] = _dot(hh, wd_ref[...])


def _ab_tables(h, ws, wd, block):
  full = lambda i: (0, 0)
  return pl.pallas_call(
      _ab_body,
      grid=(_N // block,),
      in_specs=[
          pl.BlockSpec((block, _H), lambda i: (i, 0)),
          pl.BlockSpec((_H, _H), full),
          pl.BlockSpec((_H, _H), full),
      ],
      out_specs=pl.BlockSpec((2, block, _H), lambda i: (0, i, 0)),
      out_shape=jax.ShapeDtypeStruct((2, _N, _H), _F32),
  )(h, ws, wd)


def _edge_body(e_ref, ga_ref, gb_ref, w1_ref, b1_ref, w2_ref, b2_ref, g_ref,
               be_ref, o_ref):
  e = e_ref[...]
  pre = _dot(e, w1_ref[...]) + ga_ref[...] + gb_ref[...] + b1_ref[...]
  t = jnp.maximum(pre, 0.0)
  u = _dot(t, w2_ref[...]) + b2_ref[...]
  o_ref[...] = e + _ln(u, g_ref[...], be_ref[...])


def _edge_step(e, gathered, w1e, b1, w2, b2, g, be, block):
  full = lambda i: (0, 0)
  nblk = _E // block
  return pl.pallas_call(
      _edge_body,
      grid=(nblk,),
      in_specs=[
          pl.BlockSpec((block, _H), lambda i: (i, 0)),
          pl.BlockSpec((block, _H), lambda i: (i, 0)),
          pl.BlockSpec((block, _H), lambda i: (i + nblk, 0)),
          pl.BlockSpec((_H, _H), full),
          pl.BlockSpec((1, _H), full),
          pl.BlockSpec((_H, _H), full),
          pl.BlockSpec((1, _H), full),
          pl.BlockSpec((1, _H), full),
          pl.BlockSpec((1, _H), full),
      ],
      out_specs=pl.BlockSpec((block, _H), lambda i: (i, 0)),
      out_shape=jax.ShapeDtypeStruct((_E, _H), _F32),
  )(e, gathered, gathered, w1e, b1.reshape(1, -1), w2, b2.reshape(1, -1),
    g.reshape(1, -1), be.reshape(1, -1))


def _node_body(h_ref, p_ref, w1h_ref, w1a_ref, b1_ref, w2_ref, b2_ref, g_ref,
               be_ref, o_ref):
  hh = h_ref[...]
  agg = p_ref[0] + p_ref[1]
  pre = _dot(hh, w1h_ref[...]) + _dot(agg, w1a_ref[...]) + b1_ref[...]
  t = jnp.maximum(pre, 0.0)
  u = _dot(t, w2_ref[...]) + b2_ref[...]
  o_ref[...] = hh + _ln(u, g_ref[...], be_ref[...])


def _node_step(h, parts, w1h, w1a, b1, w2, b2, g, be, block):
  full = lambda i: (0, 0)
  return pl.pallas_call(
      _node_body,
      grid=(_N // block,),
      in_specs=[
          pl.BlockSpec((block, _H), lambda i: (i, 0)),
          pl.BlockSpec((2, block, _H), lambda i: (0, i, 0)),
          pl.BlockSpec((_H, _H), full),
          pl.BlockSpec((_H, _H), full),
          pl.BlockSpec((1, _H), full),
          pl.BlockSpec((_H, _H), full),
          pl.BlockSpec((1, _H), full),
          pl.BlockSpec((1, _H), full),
          pl.BlockSpec((1, _H), full),
      ],
      out_specs=pl.BlockSpec((block, _H), lambda i: (i, 0)),
      out_shape=jax.ShapeDtypeStruct((_N, _H), _F32),
  )(h, parts, w1h, w1a, b1.reshape(1, -1), w2, b2.reshape(1, -1),
    g.reshape(1, -1), be.reshape(1, -1))


def _dec_body(h_ref, w1_ref, b1_ref, w2_ref, b2_ref, o_ref):
  t = jnp.maximum(_dot(h_ref[...], w1_ref[...]) + b1_ref[...], 0.0)
  o_ref[...] = _dot(t, w2_ref[...]) + b2_ref[...]


def _decode(h, w1, b1, w2p, b2p, block):
  full = lambda i: (0, 0)
  return pl.pallas_call(
      _dec_body,
      grid=(_N // block,),
      in_specs=[
          pl.BlockSpec((block, _H), lambda i: (i, 0)),
          pl.BlockSpec((_H, _H), full),
          pl.BlockSpec((1, _H), full),
          pl.BlockSpec((_H, _H), full),
          pl.BlockSpec((1, _H), full),
      ],
      out_specs=pl.BlockSpec((block, _H), lambda i: (i, 0)),
      out_shape=jax.ShapeDtypeStruct((_N, _H), _F32),
  )(h, w1, b1.reshape(1, -1), w2p, b2p)


def _build_gather(nrows):
  per_w = nrows // _NW
  nchunks = per_w // _CHUNK
  mesh = plsc.VectorSubcoreMesh(core_axis_name="c", subcore_axis_name="s")

  @functools.partial(
      pl.kernel,
      mesh=mesh,
      out_type=jax.ShapeDtypeStruct((nrows, _H), _F32),
      scratch_types=[
          pltpu.VMEM((_CHUNK,), jnp.int32),
          pltpu.VMEM((_CHUNK, _H), _F32),
          pltpu.SemaphoreType.DMA,
      ],
  )
  def gather(table_hbm, idx_hbm, out_hbm, idx_v, rows_v, sem):
    cid = lax.axis_index("c")
    sid = lax.axis_index("s")
    wid = cid * _NS + sid

    def body(i, carry):
      off = (wid * nchunks + i) * _CHUNK
      pltpu.sync_copy(idx_hbm.at[pl.ds(off, _CHUNK)], idx_v)
      pltpu.async_copy(table_hbm.at[idx_v], rows_v, sem).wait()
      pltpu.sync_copy(rows_v, out_hbm.at[pl.ds(off, _CHUNK)])
      return carry

    lax.fori_loop(0, nchunks, body, 0)

  return gather


def _build_scatter_add():
  per_w = _E // _NW
  nchunks = per_w // _CHUNK
  mesh = plsc.VectorSubcoreMesh(core_axis_name="c", subcore_axis_name="s")

  @functools.partial(
      pl.kernel,
      mesh=mesh,
      out_type=jax.ShapeDtypeStruct((_NC, _N, _H), _F32),
      scratch_types=[
          pltpu.VMEM((_CHUNK,), jnp.int32),
          pltpu.VMEM((_CHUNK, _H), _F32),
          pltpu.VMEM_SHARED((_N, _H), _F32),
      ],
  )
  def scatter(vals_hbm, idx_hbm, zeros_hbm, out_hbm, idx_v, rows_v, acc):
    cid = lax.axis_index("c")
    sid = lax.axis_index("s")
    wid = cid * _NS + sid

    @pl.when(sid == 0)
    def _zero():
      pltpu.sync_copy(zeros_hbm, acc)

    plsc.subcore_barrier()

    def body(i, carry):
      off = (wid * nchunks + i) * _CHUNK
      pltpu.sync_copy(idx_hbm.at[pl.ds(off, _CHUNK)], idx_v)
      pltpu.sync_copy(vals_hbm.at[pl.ds(off, _CHUNK)], rows_v)
      pltpu.sync_copy(rows_v, acc.at[idx_v], add=True)
      return carry

    lax.fori_loop(0, nchunks, body, 0)
    plsc.subcore_barrier()

    @pl.when(sid == 0)
    def _emit():
      pltpu.sync_copy(acc, out_hbm.at[cid])

  return scatter


def kernel(x, edge_index, edge_attr, enc_n_W1, enc_n_b1, enc_n_W2, enc_n_b2,
           enc_n_g, enc_n_be, enc_e_W1, enc_e_b1, enc_e_W2, enc_e_b2, enc_e_g,
           enc_e_be, pe_W1, pe_b1, pe_W2, pe_b2, pe_g, pe_be, pn_W1, pn_b1,
           pn_W2, pn_b2, pn_g, pn_be, dec_W1, dec_b1, dec_W2, dec_b2):
  src = edge_index[0].astype(jnp.int32)
  dst = edge_index[1].astype(jnp.int32)

  h = _mlp_ln(x, enc_n_W1, enc_n_b1, enc_n_W2, enc_n_b2, enc_n_g, enc_n_be,
              block=1000)
  e = _mlp_ln(edge_attr, enc_e_W1, enc_e_b1, enc_e_W2, enc_e_b2, enc_e_g,
              enc_e_be, block=2000)

  idx2 = jnp.concatenate([src, dst + _N])
  zeros = jnp.zeros((_N, _H), _F32)

  gather = _build_gather(2 * _E)
  scatter = _build_scatter_add()

  for s in range(pe_W1.shape[0]):
    w1 = pe_W1[s]
    ab = _ab_tables(h, w1[_H:2 * _H], w1[2 * _H:], block=1000)
    g2 = gather(ab.reshape(2 * _N, _H), idx2)
    e = _edge_step(e, g2, w1[:_H], pe_b1[s], pe_W2[s], pe_b2[s], pe_g[s],
                   pe_be[s], block=2000)
    parts = scatter(e, dst, zeros)
    h = _node_step(h, parts, pn_W1[s][:_H], pn_W1[s][_H:], pn_b1[s], pn_W2[s],
                   pn_b2[s], pn_g[s], pn_be[s], block=1000)

  out = _decode(h, dec_W1, dec_b1, jnp.pad(dec_W2, ((0, 0), (0, _H - 3))),
                jnp.pad(dec_b2, (0, _H - 3)).reshape(1, -1), block=1000)
  return out[:, :3]


# speedup vs baseline: 3.5438x; 1.4235x over previous
import functools

import jax
import jax.numpy as jnp
from jax import lax
from jax.experimental import pallas as pl
from jax.experimental.pallas import tpu as pltpu
from jax.experimental.pallas import tpu_sc as plsc

_N = 10000
_E = 320000
_H = 128
_NC = 2
_NS = 16
_NW = _NC * _NS
_CHUNK = 80

_F32 = jnp.float32


def _ln(u, g, b):
  mu = jnp.mean(u, axis=-1, keepdims=True)
  d = u - mu
  var = jnp.mean(d * d, axis=-1, keepdims=True)
  return d / jnp.sqrt(var + 1e-5) * g + b


def _dot(a, b):
  return jnp.dot(a, b, preferred_element_type=_F32)


def _mlp_ln_body(x_ref, w1_ref, b1_ref, w2_ref, b2_ref, g_ref, be_ref, o_ref):
  t = jnp.maximum(_dot(x_ref[...], w1_ref[...]) + b1_ref[...], 0.0)
  u = _dot(t, w2_ref[...]) + b2_ref[...]
  o_ref[...] = _ln(u, g_ref[...], be_ref[...])


def _mlp_ln(x, w1, b1, w2, b2, g, be, block):
  n, fin = x.shape
  h = w1.shape[1]
  full = lambda i: (0, 0)
  return pl.pallas_call(
      _mlp_ln_body,
      grid=(n // block,),
      in_specs=[
          pl.BlockSpec((block, fin), lambda i: (i, 0)),
          pl.BlockSpec((fin, h), full),
          pl.BlockSpec((1, h), full),
          pl.BlockSpec((h, h), full),
          pl.BlockSpec((1, h), full),
          pl.BlockSpec((1, h), full),
          pl.BlockSpec((1, h), full),
      ],
      out_specs=pl.BlockSpec((block, h), lambda i: (i, 0)),
      out_shape=jax.ShapeDtypeStruct((n, h), _F32),
  )(x, w1, b1.reshape(1, -1), w2, b2.reshape(1, -1), g.reshape(1, -1),
    be.reshape(1, -1))


def _ab_body(h_ref, ws_ref, wd_ref, o_ref):
  hh = h_ref[...]
  o_ref[0] = _dot(hh, ws_ref[...])
  o_ref[1] = _dot(hh, wd_ref[...])


def _ab_tables(h, ws, wd, block):
  full = lambda i: (0, 0)
  return pl.pallas_call(
      _ab_body,
      grid=(_N // block,),
      in_specs=[
          pl.BlockSpec((block, _H), lambda i: (i, 0)),
          pl.BlockSpec((_H, _H), full),
          pl.BlockSpec((_H, _H), full),
      ],
      out_specs=pl.BlockSpec((2, block, _H), lambda i: (0, i, 0)),
      out_shape=jax.ShapeDtypeStruct((2, _N, _H), _F32),
  )(h, ws, wd)


def _edge_body(e_ref, ga_ref, gb_ref, w1_ref, b1_ref, w2_ref, b2_ref, g_ref,
               be_ref, o_ref):
  e = e_ref[...]
  pre = _dot(e, w1_ref[...]) + ga_ref[...] + gb_ref[...] + b1_ref[...]
  t = jnp.maximum(pre, 0.0)
  u = _dot(t, w2_ref[...]) + b2_ref[...]
  o_ref[...] = e + _ln(u, g_ref[...], be_ref[...])


def _edge_step(e, gathered, w1e, b1, w2, b2, g, be, block):
  full = lambda i: (0, 0)
  nblk = _E // block
  return pl.pallas_call(
      _edge_body,
      grid=(nblk,),
      in_specs=[
          pl.BlockSpec((block, _H), lambda i: (i, 0)),
          pl.BlockSpec((block, _H), lambda i: (i, 0)),
          pl.BlockSpec((block, _H), lambda i: (i + nblk, 0)),
          pl.BlockSpec((_H, _H), full),
          pl.BlockSpec((1, _H), full),
          pl.BlockSpec((_H, _H), full),
          pl.BlockSpec((1, _H), full),
          pl.BlockSpec((1, _H), full),
          pl.BlockSpec((1, _H), full),
      ],
      out_specs=pl.BlockSpec((block, _H), lambda i: (i, 0)),
      out_shape=jax.ShapeDtypeStruct((_E, _H), _F32),
  )(e, gathered, gathered, w1e, b1.reshape(1, -1), w2, b2.reshape(1, -1),
    g.reshape(1, -1), be.reshape(1, -1))


def _node_body(h_ref, p_ref, w1h_ref, w1a_ref, b1_ref, w2_ref, b2_ref, g_ref,
               be_ref, o_ref):
  hh = h_ref[...]
  agg = p_ref[0] + p_ref[1]
  pre = _dot(hh, w1h_ref[...]) + _dot(agg, w1a_ref[...]) + b1_ref[...]
  t = jnp.maximum(pre, 0.0)
  u = _dot(t, w2_ref[...]) + b2_ref[...]
  o_ref[...] = hh + _ln(u, g_ref[...], be_ref[...])


def _node_step(h, parts, w1h, w1a, b1, w2, b2, g, be, block):
  full = lambda i: (0, 0)
  return pl.pallas_call(
      _node_body,
      grid=(_N // block,),
      in_specs=[
          pl.BlockSpec((block, _H), lambda i: (i, 0)),
          pl.BlockSpec((2, block, _H), lambda i: (0, i, 0)),
          pl.BlockSpec((_H, _H), full),
          pl.BlockSpec((_H, _H), full),
          pl.BlockSpec((1, _H), full),
          pl.BlockSpec((_H, _H), full),
          pl.BlockSpec((1, _H), full),
          pl.BlockSpec((1, _H), full),
          pl.BlockSpec((1, _H), full),
      ],
      out_specs=pl.BlockSpec((block, _H), lambda i: (i, 0)),
      out_shape=jax.ShapeDtypeStruct((_N, _H), _F32),
  )(h, parts, w1h, w1a, b1.reshape(1, -1), w2, b2.reshape(1, -1),
    g.reshape(1, -1), be.reshape(1, -1))


def _dec_body(h_ref, w1_ref, b1_ref, w2_ref, b2_ref, o_ref):
  t = jnp.maximum(_dot(h_ref[...], w1_ref[...]) + b1_ref[...], 0.0)
  o_ref[...] = _dot(t, w2_ref[...]) + b2_ref[...]


def _decode(h, w1, b1, w2p, b2p, block):
  full = lambda i: (0, 0)
  return pl.pallas_call(
      _dec_body,
      grid=(_N // block,),
      in_specs=[
          pl.BlockSpec((block, _H), lambda i: (i, 0)),
          pl.BlockSpec((_H, _H), full),
          pl.BlockSpec((1, _H), full),
          pl.BlockSpec((_H, _H), full),
          pl.BlockSpec((1, _H), full),
      ],
      out_specs=pl.BlockSpec((block, _H), lambda i: (i, 0)),
      out_shape=jax.ShapeDtypeStruct((_N, _H), _F32),
  )(h, w1, b1.reshape(1, -1), w2p, b2p)


def _build_gather(nrows):
  per_w = nrows // _NW
  nchunks = per_w // _CHUNK
  npair = nchunks // 2
  mesh = plsc.VectorSubcoreMesh(core_axis_name="c", subcore_axis_name="s")

  @functools.partial(
      pl.kernel,
      mesh=mesh,
      out_type=jax.ShapeDtypeStruct((nrows, _H), _F32),
      scratch_types=[
          pltpu.VMEM((per_w,), jnp.int32),
          pltpu.VMEM((_CHUNK, _H), _F32),
          pltpu.VMEM((_CHUNK, _H), _F32),
          pltpu.SemaphoreType.DMA,
          pltpu.SemaphoreType.DMA,
          pltpu.SemaphoreType.DMA,
          pltpu.SemaphoreType.DMA,
      ],
  )
  def gather(table_hbm, idx_hbm, out_hbm, idx_all, rows0, rows1, sg0, sg1,
             so0, so1):
    cid = lax.axis_index("c")
    sid = lax.axis_index("s")
    wid = cid * _NS + sid
    base = wid * per_w
    pltpu.sync_copy(idx_hbm.at[pl.ds(base, per_w)], idx_all)

    def body(j, carry):
      i0 = 2 * j
      i1 = i0 + 1

      @pl.when(j > 0)
      def _drain_prev_stores():
        pltpu.make_async_copy(rows0, out_hbm.at[pl.ds(0, _CHUNK)], so0).wait()
        pltpu.make_async_copy(rows1, out_hbm.at[pl.ds(0, _CHUNK)], so1).wait()

      g0 = pltpu.async_copy(
          table_hbm.at[idx_all.at[pl.ds(i0 * _CHUNK, _CHUNK)]], rows0, sg0)
      g1 = pltpu.async_copy(
          table_hbm.at[idx_all.at[pl.ds(i1 * _CHUNK, _CHUNK)]], rows1, sg1)
      g0.wait()
      pltpu.async_copy(rows0, out_hbm.at[pl.ds(base + i0 * _CHUNK, _CHUNK)],
                       so0)
      g1.wait()
      pltpu.async_copy(rows1, out_hbm.at[pl.ds(base + i1 * _CHUNK, _CHUNK)],
                       so1)
      return carry

    lax.fori_loop(0, npair, body, 0)
    pltpu.make_async_copy(rows0, out_hbm.at[pl.ds(0, _CHUNK)], so0).wait()
    pltpu.make_async_copy(rows1, out_hbm.at[pl.ds(0, _CHUNK)], so1).wait()

  return gather


def _build_scatter_add():
  per_w = _E // _NW
  nchunks = per_w // _CHUNK
  npair = nchunks // 2
  tail = nchunks - 2 * npair
  mesh = plsc.VectorSubcoreMesh(core_axis_name="c", subcore_axis_name="s")

  @functools.partial(
      pl.kernel,
      mesh=mesh,
      out_type=jax.ShapeDtypeStruct((_NC, _N, _H), _F32),
      scratch_types=[
          pltpu.VMEM((_CHUNK,), jnp.int32),
          pltpu.VMEM((_CHUNK,), jnp.int32),
          pltpu.VMEM((_CHUNK, _H), _F32),
          pltpu.VMEM((_CHUNK, _H), _F32),
          pltpu.VMEM_SHARED((_N, _H), _F32),
          pltpu.SemaphoreType.DMA,
          pltpu.SemaphoreType.DMA,
          pltpu.SemaphoreType.DMA,
          pltpu.SemaphoreType.DMA,
          pltpu.SemaphoreType.DMA,
          pltpu.SemaphoreType.DMA,
      ],
  )
  def scatter(vals_hbm, idx_hbm, zeros_hbm, out_hbm, idx0, idx1, rows0, rows1,
              acc, si0, si1, sv0, sv1, sa0, sa1):
    cid = lax.axis_index("c")
    sid = lax.axis_index("s")
    wid = cid * _NS + sid

    @pl.when(sid == 0)
    def _zero():
      pltpu.sync_copy(zeros_hbm, acc)

    plsc.subcore_barrier()

    def body(j, carry):
      i0 = 2 * j
      i1 = i0 + 1
      off0 = (wid * nchunks + i0) * _CHUNK
      off1 = (wid * nchunks + i1) * _CHUNK

      @pl.when(j > 0)
      def _drain_prev_adds():
        pltpu.make_async_copy(rows0, acc.at[idx0], sa0).wait()
        pltpu.make_async_copy(rows1, acc.at[idx1], sa1).wait()

      a0 = pltpu.async_copy(idx_hbm.at[pl.ds(off0, _CHUNK)], idx0, si0)
      v0 = pltpu.async_copy(vals_hbm.at[pl.ds(off0, _CHUNK)], rows0, sv0)
      a1 = pltpu.async_copy(idx_hbm.at[pl.ds(off1, _CHUNK)], idx1, si1)
      v1 = pltpu.async_copy(vals_hbm.at[pl.ds(off1, _CHUNK)], rows1, sv1)
      a0.wait()
      v0.wait()
      pltpu.async_copy(rows0, acc.at[idx0], sa0, add=True)
      a1.wait()
      v1.wait()
      pltpu.async_copy(rows1, acc.at[idx1], sa1, add=True)
      return carry

    lax.fori_loop(0, npair, body, 0)
    pltpu.make_async_copy(rows0, acc.at[idx0], sa0).wait()
    pltpu.make_async_copy(rows1, acc.at[idx1], sa1).wait()

    if tail:
      off = (wid * nchunks + 2 * npair) * _CHUNK
      pltpu.sync_copy(idx_hbm.at[pl.ds(off, _CHUNK)], idx0)
      pltpu.sync_copy(vals_hbm.at[pl.ds(off, _CHUNK)], rows0)
      pltpu.sync_copy(rows0, acc.at[idx0], add=True)

    plsc.subcore_barrier()

    @pl.when(sid == 0)
    def _emit():
      pltpu.sync_copy(acc, out_hbm.at[cid])

  return scatter


def kernel(x, edge_index, edge_attr, enc_n_W1, enc_n_b1, enc_n_W2, enc_n_b2,
           enc_n_g, enc_n_be, enc_e_W1, enc_e_b1, enc_e_W2, enc_e_b2, enc_e_g,
           enc_e_be, pe_W1, pe_b1, pe_W2, pe_b2, pe_g, pe_be, pn_W1, pn_b1,
           pn_W2, pn_b2, pn_g, pn_be, dec_W1, dec_b1, dec_W2, dec_b2):
  src = edge_index[0].astype(jnp.int32)
  dst = edge_index[1].astype(jnp.int32)

  h = _mlp_ln(x, enc_n_W1, enc_n_b1, enc_n_W2, enc_n_b2, enc_n_g, enc_n_be,
              block=1000)
  e = _mlp_ln(edge_attr, enc_e_W1, enc_e_b1, enc_e_W2, enc_e_b2, enc_e_g,
              enc_e_be, block=2000)

  idx2 = jnp.concatenate([src, dst + _N])
  zeros = jnp.zeros((_N, _H), _F32)

  gather = _build_gather(2 * _E)
  scatter = _build_scatter_add()

  for s in range(pe_W1.shape[0]):
    w1 = pe_W1[s]
    ab = _ab_tables(h, w1[_H:2 * _H], w1[2 * _H:], block=1000)
    g2 = gather(ab.reshape(2 * _N, _H), idx2)
    e = _edge_step(e, g2, w1[:_H], pe_b1[s], pe_W2[s], pe_b2[s], pe_g[s],
                   pe_be[s], block=2000)
    parts = scatter(e, dst, zeros)
    h = _node_step(h, parts, pn_W1[s][:_H], pn_W1[s][_H:], pn_b1[s], pn_W2[s],
                   pn_b2[s], pn_g[s], pn_be[s], block=1000)

  out = _decode(h, dec_W1, dec_b1, jnp.pad(dec_W2, ((0, 0), (0, _H - 3))),
                jnp.pad(dec_b2, (0, _H - 3)).reshape(1, -1), block=1000)
  return out[:, :3]


# speedup vs baseline: 3.9709x; 1.1205x over previous
import functools

import jax
import jax.numpy as jnp
from jax import lax
from jax.experimental import pallas as pl
from jax.experimental.pallas import tpu as pltpu
from jax.experimental.pallas import tpu_sc as plsc

_N = 10000
_E = 320000
_H = 128
_NC = 2
_NS = 16
_NW = _NC * _NS
_CHUNK = 80

_F32 = jnp.float32


def _ln(u, g, b):
  mu = jnp.mean(u, axis=-1, keepdims=True)
  d = u - mu
  var = jnp.mean(d * d, axis=-1, keepdims=True)
  return d / jnp.sqrt(var + 1e-5) * g + b


def _dot(a, b):
  return jnp.dot(a, b, preferred_element_type=_F32)


def _mlp_ln_body(x_ref, w1_ref, b1_ref, w2_ref, b2_ref, g_ref, be_ref, o_ref):
  t = jnp.maximum(_dot(x_ref[...], w1_ref[...]) + b1_ref[...], 0.0)
  u = _dot(t, w2_ref[...]) + b2_ref[...]
  o_ref[...] = _ln(u, g_ref[...], be_ref[...])


def _mlp_ln(x, w1, b1, w2, b2, g, be, block):
  n, fin = x.shape
  h = w1.shape[1]
  full = lambda i: (0, 0)
  return pl.pallas_call(
      _mlp_ln_body,
      grid=(n // block,),
      in_specs=[
          pl.BlockSpec((block, fin), lambda i: (i, 0)),
          pl.BlockSpec((fin, h), full),
          pl.BlockSpec((1, h), full),
          pl.BlockSpec((h, h), full),
          pl.BlockSpec((1, h), full),
          pl.BlockSpec((1, h), full),
          pl.BlockSpec((1, h), full),
      ],
      out_specs=pl.BlockSpec((block, h), lambda i: (i, 0)),
      out_shape=jax.ShapeDtypeStruct((n, h), _F32),
  )(x, w1, b1.reshape(1, -1), w2, b2.reshape(1, -1), g.reshape(1, -1),
    be.reshape(1, -1))


def _ab_body(h_ref, ws_ref, wd_ref, o_ref):
  hh = h_ref[...]
  o_ref[0] = _dot(hh, ws_ref[...])
  o_ref[1] = _dot(hh, wd_ref[...])


def _ab_tables(h, ws, wd, block):
  full = lambda i: (0, 0)
  return pl.pallas_call(
      _ab_body,
      grid=(_N // block,),
      in_specs=[
          pl.BlockSpec((block, _H), lambda i: (i, 0)),
          pl.BlockSpec((_H, _H), full),
          pl.BlockSpec((_H, _H), full),
      ],
      out_specs=pl.BlockSpec((2, block, _H), lambda i: (0, i, 0)),
      out_shape=jax.ShapeDtypeStruct((2, _N, _H), _F32),
  )(h, ws, wd)


def _edge_body(e_ref, ga_ref, gb_ref, w1_ref, b1_ref, w2_ref, b2_ref, g_ref,
               be_ref, o_ref):
  e = e_ref[...]
  pre = _dot(e, w1_ref[...]) + ga_ref[...] + gb_ref[...] + b1_ref[...]
  t = jnp.maximum(pre, 0.0)
  u = _dot(t, w2_ref[...]) + b2_ref[...]
  o_ref[...] = e + _ln(u, g_ref[...], be_ref[...])


def _edge_step(e, gathered, w1e, b1, w2, b2, g, be, block):
  full = lambda i: (0, 0)
  nblk = _E // block
  return pl.pallas_call(
      _edge_body,
      grid=(nblk,),
      in_specs=[
          pl.BlockSpec((block, _H), lambda i: (i, 0)),
          pl.BlockSpec((block, _H), lambda i: (i, 0)),
          pl.BlockSpec((block, _H), lambda i: (i + nblk, 0)),
          pl.BlockSpec((_H, _H), full),
          pl.BlockSpec((1, _H), full),
          pl.BlockSpec((_H, _H), full),
          pl.BlockSpec((1, _H), full),
          pl.BlockSpec((1, _H), full),
          pl.BlockSpec((1, _H), full),
      ],
      out_specs=pl.BlockSpec((block, _H), lambda i: (i, 0)),
      out_shape=jax.ShapeDtypeStruct((_E, _H), _F32),
  )(e, gathered, gathered, w1e, b1.reshape(1, -1), w2, b2.reshape(1, -1),
    g.reshape(1, -1), be.reshape(1, -1))


def _node_body(h_ref, p_ref, w1h_ref, w1a_ref, b1_ref, w2_ref, b2_ref, g_ref,
               be_ref, o_ref):
  hh = h_ref[...]
  agg = p_ref[0] + p_ref[1]
  pre = _dot(hh, w1h_ref[...]) + _dot(agg, w1a_ref[...]) + b1_ref[...]
  t = jnp.maximum(pre, 0.0)
  u = _dot(t, w2_ref[...]) + b2_ref[...]
  o_ref[...] = hh + _ln(u, g_ref[...], be_ref[...])


def _node_step(h, parts, w1h, w1a, b1, w2, b2, g, be, block):
  full = lambda i: (0, 0)
  return pl.pallas_call(
      _node_body,
      grid=(_N // block,),
      in_specs=[
          pl.BlockSpec((block, _H), lambda i: (i, 0)),
          pl.BlockSpec((2, block, _H), lambda i: (0, i, 0)),
          pl.BlockSpec((_H, _H), full),
          pl.BlockSpec((_H, _H), full),
          pl.BlockSpec((1, _H), full),
          pl.BlockSpec((_H, _H), full),
          pl.BlockSpec((1, _H), full),
          pl.BlockSpec((1, _H), full),
          pl.BlockSpec((1, _H), full),
      ],
      out_specs=pl.BlockSpec((block, _H), lambda i: (i, 0)),
      out_shape=jax.ShapeDtypeStruct((_N, _H), _F32),
  )(h, parts, w1h, w1a, b1.reshape(1, -1), w2, b2.reshape(1, -1),
    g.reshape(1, -1), be.reshape(1, -1))


def _dec_body(h_ref, w1_ref, b1_ref, w2_ref, b2_ref, o_ref):
  t = jnp.maximum(_dot(h_ref[...], w1_ref[...]) + b1_ref[...], 0.0)
  o_ref[...] = _dot(t, w2_ref[...]) + b2_ref[...]


def _decode(h, w1, b1, w2p, b2p, block):
  full = lambda i: (0, 0)
  return pl.pallas_call(
      _dec_body,
      grid=(_N // block,),
      in_specs=[
          pl.BlockSpec((block, _H), lambda i: (i, 0)),
          pl.BlockSpec((_H, _H), full),
          pl.BlockSpec((1, _H), full),
          pl.BlockSpec((_H, _H), full),
          pl.BlockSpec((1, _H), full),
      ],
      out_specs=pl.BlockSpec((block, _H), lambda i: (i, 0)),
      out_shape=jax.ShapeDtypeStruct((_N, _H), _F32),
  )(h, w1, b1.reshape(1, -1), w2p, b2p)


def _build_gather_spmem():
  per_w = _E // _NS
  nchunks = per_w // _CHUNK
  npair = nchunks // 2
  mesh = plsc.VectorSubcoreMesh(core_axis_name="c", subcore_axis_name="s")

  @functools.partial(
      pl.kernel,
      mesh=mesh,
      out_type=jax.ShapeDtypeStruct((2 * _E, _H), _F32),
      scratch_types=[
          pltpu.VMEM((per_w,), jnp.int32),
          pltpu.VMEM((_CHUNK, _H), _F32),
          pltpu.VMEM((_CHUNK, _H), _F32),
          pltpu.VMEM_SHARED((_N, _H), _F32),
          pltpu.SemaphoreType.DMA,
          pltpu.SemaphoreType.DMA,
          pltpu.SemaphoreType.DMA,
          pltpu.SemaphoreType.DMA,
      ],
  )
  def gather(ab_hbm, idx_hbm, out_hbm, idx_all, rows0, rows1, tbl, sg0, sg1,
             so0, so1):
    cid = lax.axis_index("c")
    sid = lax.axis_index("s")
    base = cid * _E + sid * per_w

    @pl.when(sid == 0)
    def _stage_table():
      pltpu.sync_copy(ab_hbm.at[cid], tbl)

    pltpu.sync_copy(idx_hbm.at[pl.ds(base, per_w)], idx_all)
    plsc.subcore_barrier()

    def body(j, carry):
      i0 = 2 * j
      i1 = i0 + 1

      @pl.when(j > 0)
      def _drain_prev_stores():
        pltpu.make_async_copy(rows0, out_hbm.at[pl.ds(0, _CHUNK)], so0).wait()
        pltpu.make_async_copy(rows1, out_hbm.at[pl.ds(0, _CHUNK)], so1).wait()

      g0 = pltpu.async_copy(
          tbl.at[idx_all.at[pl.ds(i0 * _CHUNK, _CHUNK)]], rows0, sg0)
      g1 = pltpu.async_copy(
          tbl.at[idx_all.at[pl.ds(i1 * _CHUNK, _CHUNK)]], rows1, sg1)
      g0.wait()
      pltpu.async_copy(rows0, out_hbm.at[pl.ds(base + i0 * _CHUNK, _CHUNK)],
                       so0)
      g1.wait()
      pltpu.async_copy(rows1, out_hbm.at[pl.ds(base + i1 * _CHUNK, _CHUNK)],
                       so1)
      return carry

    lax.fori_loop(0, npair, body, 0)
    pltpu.make_async_copy(rows0, out_hbm.at[pl.ds(0, _CHUNK)], so0).wait()
    pltpu.make_async_copy(rows1, out_hbm.at[pl.ds(0, _CHUNK)], so1).wait()

  return gather


def _build_scatter_add():
  per_w = _E // _NW
  nchunks = per_w // _CHUNK
  npair = nchunks // 2
  tail = nchunks - 2 * npair
  mesh = plsc.VectorSubcoreMesh(core_axis_name="c", subcore_axis_name="s")

  @functools.partial(
      pl.kernel,
      mesh=mesh,
      out_type=jax.ShapeDtypeStruct((_NC, _N, _H), _F32),
      scratch_types=[
          pltpu.VMEM((_CHUNK,), jnp.int32),
          pltpu.VMEM((_CHUNK,), jnp.int32),
          pltpu.VMEM((_CHUNK, _H), _F32),
          pltpu.VMEM((_CHUNK, _H), _F32),
          pltpu.VMEM_SHARED((_N, _H), _F32),
          pltpu.SemaphoreType.DMA,
          pltpu.SemaphoreType.DMA,
          pltpu.SemaphoreType.DMA,
          pltpu.SemaphoreType.DMA,
          pltpu.SemaphoreType.DMA,
          pltpu.SemaphoreType.DMA,
      ],
  )
  def scatter(vals_hbm, idx_hbm, zeros_hbm, out_hbm, idx0, idx1, rows0, rows1,
              acc, si0, si1, sv0, sv1, sa0, sa1):
    cid = lax.axis_index("c")
    sid = lax.axis_index("s")
    wid = cid * _NS + sid

    @pl.when(sid == 0)
    def _zero():
      pltpu.sync_copy(zeros_hbm, acc)

    plsc.subcore_barrier()

    def body(j, carry):
      i0 = 2 * j
      i1 = i0 + 1
      off0 = (wid * nchunks + i0) * _CHUNK
      off1 = (wid * nchunks + i1) * _CHUNK

      @pl.when(j > 0)
      def _drain_prev_adds():
        pltpu.make_async_copy(rows0, acc.at[idx0], sa0).wait()
        pltpu.make_async_copy(rows1, acc.at[idx1], sa1).wait()

      a0 = pltpu.async_copy(idx_hbm.at[pl.ds(off0, _CHUNK)], idx0, si0)
      v0 = pltpu.async_copy(vals_hbm.at[pl.ds(off0, _CHUNK)], rows0, sv0)
      a1 = pltpu.async_copy(idx_hbm.at[pl.ds(off1, _CHUNK)], idx1, si1)
      v1 = pltpu.async_copy(vals_hbm.at[pl.ds(off1, _CHUNK)], rows1, sv1)
      a0.wait()
      v0.wait()
      pltpu.async_copy(rows0, acc.at[idx0], sa0, add=True)
      a1.wait()
      v1.wait()
      pltpu.async_copy(rows1, acc.at[idx1], sa1, add=True)
      return carry

    lax.fori_loop(0, npair, body, 0)
    pltpu.make_async_copy(rows0, acc.at[idx0], sa0).wait()
    pltpu.make_async_copy(rows1, acc.at[idx1], sa1).wait()

    if tail:
      off = (wid * nchunks + 2 * npair) * _CHUNK
      pltpu.sync_copy(idx_hbm.at[pl.ds(off, _CHUNK)], idx0)
      pltpu.sync_copy(vals_hbm.at[pl.ds(off, _CHUNK)], rows0)
      pltpu.sync_copy(rows0, acc.at[idx0], add=True)

    plsc.subcore_barrier()

    @pl.when(sid == 0)
    def _emit():
      pltpu.sync_copy(acc, out_hbm.at[cid])

  return scatter


def kernel(x, edge_index, edge_attr, enc_n_W1, enc_n_b1, enc_n_W2, enc_n_b2,
           enc_n_g, enc_n_be, enc_e_W1, enc_e_b1, enc_e_W2, enc_e_b2, enc_e_g,
           enc_e_be, pe_W1, pe_b1, pe_W2, pe_b2, pe_g, pe_be, pn_W1, pn_b1,
           pn_W2, pn_b2, pn_g, pn_be, dec_W1, dec_b1, dec_W2, dec_b2):
  src = edge_index[0].astype(jnp.int32)
  dst = edge_index[1].astype(jnp.int32)

  h = _mlp_ln(x, enc_n_W1, enc_n_b1, enc_n_W2, enc_n_b2, enc_n_g, enc_n_be,
              block=1000)
  e = _mlp_ln(edge_attr, enc_e_W1, enc_e_b1, enc_e_W2, enc_e_b2, enc_e_g,
              enc_e_be, block=2000)

  idx2 = jnp.concatenate([src, dst])
  zeros = jnp.zeros((_N, _H), _F32)

  gather = _build_gather_spmem()
  scatter = _build_scatter_add()

  for s in range(pe_W1.shape[0]):
    w1 = pe_W1[s]
    ab = _ab_tables(h, w1[_H:2 * _H], w1[2 * _H:], block=1000)
    g2 = gather(ab, idx2)
    e = _edge_step(e, g2, w1[:_H], pe_b1[s], pe_W2[s], pe_b2[s], pe_g[s],
                   pe_be[s], block=2000)
    parts = scatter(e, dst, zeros)
    h = _node_step(h, parts, pn_W1[s][:_H], pn_W1[s][_H:], pn_b1[s], pn_W2[s],
                   pn_b2[s], pn_g[s], pn_be[s], block=1000)

  out = _decode(h, dec_W1, dec_b1, jnp.pad(dec_W2, ((0, 0), (0, _H - 3))),
                jnp.pad(dec_b2, (0, _H - 3)).reshape(1, -1), block=1000)
  return out[:, :3]


# speedup vs baseline: 4.7130x; 1.1869x over previous
import functools

import jax
import jax.numpy as jnp
from jax import lax
from jax.experimental import pallas as pl
from jax.experimental.pallas import tpu as pltpu
from jax.experimental.pallas import tpu_sc as plsc

_N = 10000
_E = 320000
_H = 128
_NC = 2
_NS = 16
_NW = _NC * _NS
_CHUNK = 80

_F32 = jnp.float32


def _ln(u, g, b):
  mu = jnp.mean(u, axis=-1, keepdims=True)
  d = u - mu
  var = jnp.mean(d * d, axis=-1, keepdims=True)
  return d / jnp.sqrt(var + 1e-5) * g + b


def _dot(a, b):
  return jnp.dot(a, b, preferred_element_type=_F32)


def _mlp_ln_body(x_ref, w1_ref, b1_ref, w2_ref, b2_ref, g_ref, be_ref, o_ref):
  t = jnp.maximum(_dot(x_ref[...], w1_ref[...]) + b1_ref[...], 0.0)
  u = _dot(t, w2_ref[...]) + b2_ref[...]
  o_ref[...] = _ln(u, g_ref[...], be_ref[...])


def _mlp_ln(x, w1, b1, w2, b2, g, be, block):
  n, fin = x.shape
  h = w1.shape[1]
  full = lambda i: (0, 0)
  return pl.pallas_call(
      _mlp_ln_body,
      grid=(n // block,),
      in_specs=[
          pl.BlockSpec((block, fin), lambda i: (i, 0)),
          pl.BlockSpec((fin, h), full),
          pl.BlockSpec((1, h), full),
          pl.BlockSpec((h, h), full),
          pl.BlockSpec((1, h), full),
          pl.BlockSpec((1, h), full),
          pl.BlockSpec((1, h), full),
      ],
      out_specs=pl.BlockSpec((block, h), lambda i: (i, 0)),
      out_shape=jax.ShapeDtypeStruct((n, h), _F32),
  )(x, w1, b1.reshape(1, -1), w2, b2.reshape(1, -1), g.reshape(1, -1),
    be.reshape(1, -1))


def _ab_body(h_ref, ws_ref, wd_ref, o_ref):
  hh = h_ref[...]
  o_ref[0] = _dot(hh, ws_ref[...])
  o_ref[1] = _dot(hh, wd_ref[...])


def _ab_tables(h, ws, wd, block):
  full = lambda i: (0, 0)
  return pl.pallas_call(
      _ab_body,
      grid=(_N // block,),
      in_specs=[
          pl.BlockSpec((block, _H), lambda i: (i, 0)),
          pl.BlockSpec((_H, _H), full),
          pl.BlockSpec((_H, _H), full),
      ],
      out_specs=pl.BlockSpec((2, block, _H), lambda i: (0, i, 0)),
      out_shape=jax.ShapeDtypeStruct((2, _N, _H), _F32),
  )(h, ws, wd)


def _edge_body(e_ref, ga_ref, gb_ref, w1_ref, b1_ref, w2_ref, b2_ref, g_ref,
               be_ref, o_ref):
  e = e_ref[...]
  pre = _dot(e, w1_ref[...]) + ga_ref[0] + gb_ref[0] + b1_ref[...]
  t = jnp.maximum(pre, 0.0)
  u = _dot(t, w2_ref[...]) + b2_ref[...]
  o_ref[...] = e + _ln(u, g_ref[...], be_ref[...])


def _edge_step(e, gathered, w1e, b1, w2, b2, g, be, block):
  full = lambda i: (0, 0)
  count = e.shape[0]
  return pl.pallas_call(
      _edge_body,
      grid=(count // block,),
      in_specs=[
          pl.BlockSpec((block, _H), lambda i: (i, 0)),
          pl.BlockSpec((1, block, _H), lambda i: (0, i, 0)),
          pl.BlockSpec((1, block, _H), lambda i: (1, i, 0)),
          pl.BlockSpec((_H, _H), full),
          pl.BlockSpec((1, _H), full),
          pl.BlockSpec((_H, _H), full),
          pl.BlockSpec((1, _H), full),
          pl.BlockSpec((1, _H), full),
          pl.BlockSpec((1, _H), full),
      ],
      out_specs=pl.BlockSpec((block, _H), lambda i: (i, 0)),
      out_shape=jax.ShapeDtypeStruct((count, _H), _F32),
  )(e, gathered, gathered, w1e, b1.reshape(1, -1), w2, b2.reshape(1, -1),
    g.reshape(1, -1), be.reshape(1, -1))


def _node_body(h_ref, p_ref, q_ref, w1h_ref, w1a_ref, b1_ref, w2_ref, b2_ref,
               g_ref, be_ref, o_ref):
  hh = h_ref[...]
  agg = (p_ref[0] + p_ref[1]) + (q_ref[0] + q_ref[1])
  pre = _dot(hh, w1h_ref[...]) + _dot(agg, w1a_ref[...]) + b1_ref[...]
  t = jnp.maximum(pre, 0.0)
  u = _dot(t, w2_ref[...]) + b2_ref[...]
  o_ref[...] = hh + _ln(u, g_ref[...], be_ref[...])


def _node_step(h, parts0, parts1, w1h, w1a, b1, w2, b2, g, be, block):
  full = lambda i: (0, 0)
  return pl.pallas_call(
      _node_body,
      grid=(_N // block,),
      in_specs=[
          pl.BlockSpec((block, _H), lambda i: (i, 0)),
          pl.BlockSpec((2, block, _H), lambda i: (0, i, 0)),
          pl.BlockSpec((2, block, _H), lambda i: (0, i, 0)),
          pl.BlockSpec((_H, _H), full),
          pl.BlockSpec((_H, _H), full),
          pl.BlockSpec((1, _H), full),
          pl.BlockSpec((_H, _H), full),
          pl.BlockSpec((1, _H), full),
          pl.BlockSpec((1, _H), full),
          pl.BlockSpec((1, _H), full),
      ],
      out_specs=pl.BlockSpec((block, _H), lambda i: (i, 0)),
      out_shape=jax.ShapeDtypeStruct((_N, _H), _F32),
  )(h, parts0, parts1, w1h, w1a, b1.reshape(1, -1), w2, b2.reshape(1, -1),
    g.reshape(1, -1), be.reshape(1, -1))


def _dec_body(h_ref, w1_ref, b1_ref, w2_ref, b2_ref, o_ref):
  t = jnp.maximum(_dot(h_ref[...], w1_ref[...]) + b1_ref[...], 0.0)
  o_ref[...] = _dot(t, w2_ref[...]) + b2_ref[...]


def _decode(h, w1, b1, w2p, b2p, block):
  full = lambda i: (0, 0)
  return pl.pallas_call(
      _dec_body,
      grid=(_N // block,),
      in_specs=[
          pl.BlockSpec((block, _H), lambda i: (i, 0)),
          pl.BlockSpec((_H, _H), full),
          pl.BlockSpec((1, _H), full),
          pl.BlockSpec((_H, _H), full),
          pl.BlockSpec((1, _H), full),
      ],
      out_specs=pl.BlockSpec((block, _H), lambda i: (i, 0)),
      out_shape=jax.ShapeDtypeStruct((_N, _H), _F32),
  )(h, w1, b1.reshape(1, -1), w2p, b2p)


def _build_gather_spmem(base, count):
  per_w = count // _NS
  nchunks = per_w // _CHUNK
  npair = nchunks // 2
  tail = nchunks - 2 * npair
  mesh = plsc.VectorSubcoreMesh(core_axis_name="c", subcore_axis_name="s")

  @functools.partial(
      pl.kernel,
      mesh=mesh,
      out_type=jax.ShapeDtypeStruct((2, count, _H), _F32),
      scratch_types=[
          pltpu.VMEM((per_w,), jnp.int32),
          pltpu.VMEM((_CHUNK, _H), _F32),
          pltpu.VMEM((_CHUNK, _H), _F32),
          pltpu.VMEM_SHARED((_N, _H), _F32),
          pltpu.SemaphoreType.DMA,
          pltpu.SemaphoreType.DMA,
          pltpu.SemaphoreType.DMA,
          pltpu.SemaphoreType.DMA,
      ],
  )
  def gather(ab_hbm, idx_hbm, out_hbm, idx_all, rows0, rows1, tbl, sg0, sg1,
             so0, so1):
    cid = lax.axis_index("c")
    sid = lax.axis_index("s")
    row0 = sid * per_w
    out_c = out_hbm.at[cid]

    @pl.when(sid == 0)
    def _stage_table():
      pltpu.sync_copy(ab_hbm.at[cid], tbl)

    pltpu.sync_copy(idx_hbm.at[pl.ds(cid * _E + base + row0, per_w)], idx_all)
    plsc.subcore_barrier()

    def body(j, carry):
      i0 = 2 * j
      i1 = i0 + 1

      @pl.when(j > 0)
      def _drain_prev_stores():
        pltpu.make_async_copy(rows0, out_c.at[pl.ds(0, _CHUNK)], so0).wait()
        pltpu.make_async_copy(rows1, out_c.at[pl.ds(0, _CHUNK)], so1).wait()

      g0 = pltpu.async_copy(
          tbl.at[idx_all.at[pl.ds(i0 * _CHUNK, _CHUNK)]], rows0, sg0)
      g1 = pltpu.async_copy(
          tbl.at[idx_all.at[pl.ds(i1 * _CHUNK, _CHUNK)]], rows1, sg1)
      g0.wait()
      pltpu.async_copy(rows0, out_c.at[pl.ds(row0 + i0 * _CHUNK, _CHUNK)], so0)
      g1.wait()
      pltpu.async_copy(rows1, out_c.at[pl.ds(row0 + i1 * _CHUNK, _CHUNK)], so1)
      return carry

    lax.fori_loop(0, npair, body, 0)
    pltpu.make_async_copy(rows0, out_c.at[pl.ds(0, _CHUNK)], so0).wait()
    pltpu.make_async_copy(rows1, out_c.at[pl.ds(0, _CHUNK)], so1).wait()

    if tail:
      i = 2 * npair
      pltpu.async_copy(
          tbl.at[idx_all.at[pl.ds(i * _CHUNK, _CHUNK)]], rows0, sg0).wait()
      pltpu.sync_copy(rows0, out_c.at[pl.ds(row0 + i * _CHUNK, _CHUNK)])

  return gather


def _build_scatter_add(count):
  per_w = count // _NW
  nfull = per_w // _CHUNK
  rem = per_w - nfull * _CHUNK
  npair = nfull // 2
  odd = nfull - 2 * npair
  mesh = plsc.VectorSubcoreMesh(core_axis_name="c", subcore_axis_name="s")

  scratch = [
      pltpu.VMEM((_CHUNK,), jnp.int32),
      pltpu.VMEM((_CHUNK,), jnp.int32),
      pltpu.VMEM((_CHUNK, _H), _F32),
      pltpu.VMEM((_CHUNK, _H), _F32),
      pltpu.VMEM_SHARED((_N, _H), _F32),
      pltpu.SemaphoreType.DMA,
      pltpu.SemaphoreType.DMA,
      pltpu.SemaphoreType.DMA,
      pltpu.SemaphoreType.DMA,
      pltpu.SemaphoreType.DMA,
      pltpu.SemaphoreType.DMA,
  ]
  if rem:
    scratch += [pltpu.VMEM((rem,), jnp.int32), pltpu.VMEM((rem, _H), _F32)]

  @functools.partial(
      pl.kernel,
      mesh=mesh,
      out_type=jax.ShapeDtypeStruct((_NC, _N, _H), _F32),
      scratch_types=scratch,
  )
  def scatter(vals_hbm, idx_hbm, zeros_hbm, out_hbm, idx0, idx1, rows0, rows1,
              acc, si0, si1, sv0, sv1, sa0, sa1, *tail_bufs):
    cid = lax.axis_index("c")
    sid = lax.axis_index("s")
    wid = cid * _NS + sid
    base = wid * per_w

    @pl.when(sid == 0)
    def _zero():
      pltpu.sync_copy(zeros_hbm, acc)

    plsc.subcore_barrier()

    def body(j, carry):
      off0 = base + (2 * j) * _CHUNK
      off1 = off0 + _CHUNK

      @pl.when(j > 0)
      def _drain_prev_adds():
        pltpu.make_async_copy(rows0, acc.at[idx0], sa0).wait()
        pltpu.make_async_copy(rows1, acc.at[idx1], sa1).wait()

      a0 = pltpu.async_copy(idx_hbm.at[pl.ds(off0, _CHUNK)], idx0, si0)
      v0 = pltpu.async_copy(vals_hbm.at[pl.ds(off0, _CHUNK)], rows0, sv0)
      a1 = pltpu.async_copy(idx_hbm.at[pl.ds(off1, _CHUNK)], idx1, si1)
      v1 = pltpu.async_copy(vals_hbm.at[pl.ds(off1, _CHUNK)], rows1, sv1)
      a0.wait()
      v0.wait()
      pltpu.async_copy(rows0, acc.at[idx0], sa0, add=True)
      a1.wait()
      v1.wait()
      pltpu.async_copy(rows1, acc.at[idx1], sa1, add=True)
      return carry

    lax.fori_loop(0, npair, body, 0)
    pltpu.make_async_copy(rows0, acc.at[idx0], sa0).wait()
    pltpu.make_async_copy(rows1, acc.at[idx1], sa1).wait()

    if odd:
      off = base + 2 * npair * _CHUNK
      pltpu.sync_copy(idx_hbm.at[pl.ds(off, _CHUNK)], idx0)
      pltpu.sync_copy(vals_hbm.at[pl.ds(off, _CHUNK)], rows0)
      pltpu.sync_copy(rows0, acc.at[idx0], add=True)

    if rem:
      idxr, rowsr = tail_bufs
      off = base + nfull * _CHUNK
      pltpu.sync_copy(idx_hbm.at[pl.ds(off, rem)], idxr)
      pltpu.sync_copy(vals_hbm.at[pl.ds(off, rem)], rowsr)
      pltpu.sync_copy(rowsr, acc.at[idxr], add=True)

    plsc.subcore_barrier()

    @pl.when(sid == 0)
    def _emit():
      pltpu.sync_copy(acc, out_hbm.at[cid])

  return scatter


def kernel(x, edge_index, edge_attr, enc_n_W1, enc_n_b1, enc_n_W2, enc_n_b2,
           enc_n_g, enc_n_be, enc_e_W1, enc_e_b1, enc_e_W2, enc_e_b2, enc_e_g,
           enc_e_be, pe_W1, pe_b1, pe_W2, pe_b2, pe_g, pe_be, pn_W1, pn_b1,
           pn_W2, pn_b2, pn_g, pn_be, dec_W1, dec_b1, dec_W2, dec_b2):
  src = edge_index[0].astype(jnp.int32)
  dst = edge_index[1].astype(jnp.int32)

  half = _E // 2
  h = _mlp_ln(x, enc_n_W1, enc_n_b1, enc_n_W2, enc_n_b2, enc_n_g, enc_n_be,
              block=1000)
  e_halves = [
      _mlp_ln(edge_attr[:half], enc_e_W1, enc_e_b1, enc_e_W2, enc_e_b2,
              enc_e_g, enc_e_be, block=2000),
      _mlp_ln(edge_attr[half:], enc_e_W1, enc_e_b1, enc_e_W2, enc_e_b2,
              enc_e_g, enc_e_be, block=2000),
  ]

  idx2 = jnp.concatenate([src, dst])
  dst_halves = [dst[:half], dst[half:]]
  zeros = jnp.zeros((_N, _H), _F32)

  gathers = [_build_gather_spmem(0, half), _build_gather_spmem(half, half)]
  scatter = _build_scatter_add(half)

  for s in range(pe_W1.shape[0]):
    w1 = pe_W1[s]
    ab = _ab_tables(h, w1[_H:2 * _H], w1[2 * _H:], block=1000)
    g_halves = [gathers[0](ab, idx2), gathers[1](ab, idx2)]
    parts = []
    for k in (0, 1):
      e_halves[k] = _edge_step(e_halves[k], g_halves[k], w1[:_H], pe_b1[s],
                               pe_W2[s], pe_b2[s], pe_g[s], pe_be[s],
                               block=2000)
      parts.append(scatter(e_halves[k], dst_halves[k], zeros))
    h = _node_step(h, parts[0], parts[1], pn_W1[s][:_H], pn_W1[s][_H:],
                   pn_b1[s], pn_W2[s], pn_b2[s], pn_g[s], pn_be[s], block=1000)

  out = _decode(h, dec_W1, dec_b1, jnp.pad(dec_W2, ((0, 0), (0, _H - 3))),
                jnp.pad(dec_b2, (0, _H - 3)).reshape(1, -1), block=1000)
  return out[:, :3]


# speedup vs baseline: 4.7565x; 1.0092x over previous
import functools

import jax
import jax.numpy as jnp
from jax import lax
from jax.experimental import pallas as pl
from jax.experimental.pallas import tpu as pltpu
from jax.experimental.pallas import tpu_sc as plsc

_N = 10000
_E = 320000
_H = 128
_NC = 2
_NS = 16
_NW = _NC * _NS
_CHUNK = 128

_F32 = jnp.float32


def _ln(u, g, b):
  mu = jnp.mean(u, axis=-1, keepdims=True)
  d = u - mu
  var = jnp.mean(d * d, axis=-1, keepdims=True)
  return d / jnp.sqrt(var + 1e-5) * g + b


def _dot(a, b):
  return jnp.dot(a, b, preferred_element_type=_F32)


def _mlp_ln_body(x_ref, w1_ref, b1_ref, w2_ref, b2_ref, g_ref, be_ref, o_ref):
  t = jnp.maximum(_dot(x_ref[...], w1_ref[...]) + b1_ref[...], 0.0)
  u = _dot(t, w2_ref[...]) + b2_ref[...]
  o_ref[...] = _ln(u, g_ref[...], be_ref[...])


def _mlp_ln(x, w1, b1, w2, b2, g, be, block):
  n, fin = x.shape
  h = w1.shape[1]
  full = lambda i: (0, 0)
  return pl.pallas_call(
      _mlp_ln_body,
      grid=(n // block,),
      in_specs=[
          pl.BlockSpec((block, fin), lambda i: (i, 0)),
          pl.BlockSpec((fin, h), full),
          pl.BlockSpec((1, h), full),
          pl.BlockSpec((h, h), full),
          pl.BlockSpec((1, h), full),
          pl.BlockSpec((1, h), full),
          pl.BlockSpec((1, h), full),
      ],
      out_specs=pl.BlockSpec((block, h), lambda i: (i, 0)),
      out_shape=jax.ShapeDtypeStruct((n, h), _F32),
  )(x, w1, b1.reshape(1, -1), w2, b2.reshape(1, -1), g.reshape(1, -1),
    be.reshape(1, -1))


def _ab_body(h_ref, ws_ref, wd_ref, o_ref):
  hh = h_ref[...]
  o_ref[0] = _dot(hh, ws_ref[...])
  o_ref[1] = _dot(hh, wd_ref[...])


def _ab_tables(h, ws, wd, block):
  full = lambda i: (0, 0)
  return pl.pallas_call(
      _ab_body,
      grid=(_N // block,),
      in_specs=[
          pl.BlockSpec((block, _H), lambda i: (i, 0)),
          pl.BlockSpec((_H, _H), full),
          pl.BlockSpec((_H, _H), full),
      ],
      out_specs=pl.BlockSpec((2, block, _H), lambda i: (0, i, 0)),
      out_shape=jax.ShapeDtypeStruct((2, _N, _H), _F32),
  )(h, ws, wd)


def _edge_body(e_ref, ga_ref, gb_ref, w1_ref, b1_ref, w2_ref, b2_ref, g_ref,
               be_ref, o_ref):
  e = e_ref[...]
  pre = _dot(e, w1_ref[...]) + ga_ref[0] + gb_ref[0] + b1_ref[...]
  t = jnp.maximum(pre, 0.0)
  u = _dot(t, w2_ref[...]) + b2_ref[...]
  o_ref[...] = e + _ln(u, g_ref[...], be_ref[...])


def _edge_step(e, gathered, w1e, b1, w2, b2, g, be, block):
  full = lambda i: (0, 0)
  count = e.shape[0]
  return pl.pallas_call(
      _edge_body,
      grid=(count // block,),
      in_specs=[
          pl.BlockSpec((block, _H), lambda i: (i, 0)),
          pl.BlockSpec((1, block, _H), lambda i: (0, i, 0)),
          pl.BlockSpec((1, block, _H), lambda i: (1, i, 0)),
          pl.BlockSpec((_H, _H), full),
          pl.BlockSpec((1, _H), full),
          pl.BlockSpec((_H, _H), full),
          pl.BlockSpec((1, _H), full),
          pl.BlockSpec((1, _H), full),
          pl.BlockSpec((1, _H), full),
      ],
      out_specs=pl.BlockSpec((block, _H), lambda i: (i, 0)),
      out_shape=jax.ShapeDtypeStruct((count, _H), _F32),
  )(e, gathered, gathered, w1e, b1.reshape(1, -1), w2, b2.reshape(1, -1),
    g.reshape(1, -1), be.reshape(1, -1))


def _node_body(h_ref, p_ref, q_ref, w1h_ref, w1a_ref, b1_ref, w2_ref, b2_ref,
               g_ref, be_ref, o_ref):
  hh = h_ref[...]
  agg = (p_ref[0] + p_ref[1]) + (q_ref[0] + q_ref[1])
  pre = _dot(hh, w1h_ref[...]) + _dot(agg, w1a_ref[...]) + b1_ref[...]
  t = jnp.maximum(pre, 0.0)
  u = _dot(t, w2_ref[...]) + b2_ref[...]
  o_ref[...] = hh + _ln(u, g_ref[...], be_ref[...])


def _node_step(h, parts0, parts1, w1h, w1a, b1, w2, b2, g, be, block):
  full = lambda i: (0, 0)
  return pl.pallas_call(
      _node_body,
      grid=(_N // block,),
      in_specs=[
          pl.BlockSpec((block, _H), lambda i: (i, 0)),
          pl.BlockSpec((2, block, _H), lambda i: (0, i, 0)),
          pl.BlockSpec((2, block, _H), lambda i: (0, i, 0)),
          pl.BlockSpec((_H, _H), full),
          pl.BlockSpec((_H, _H), full),
          pl.BlockSpec((1, _H), full),
          pl.BlockSpec((_H, _H), full),
          pl.BlockSpec((1, _H), full),
          pl.BlockSpec((1, _H), full),
          pl.BlockSpec((1, _H), full),
      ],
      out_specs=pl.BlockSpec((block, _H), lambda i: (i, 0)),
      out_shape=jax.ShapeDtypeStruct((_N, _H), _F32),
  )(h, parts0, parts1, w1h, w1a, b1.reshape(1, -1), w2, b2.reshape(1, -1),
    g.reshape(1, -1), be.reshape(1, -1))


def _dec_body(h_ref, w1_ref, b1_ref, w2_ref, b2_ref, o_ref):
  t = jnp.maximum(_dot(h_ref[...], w1_ref[...]) + b1_ref[...], 0.0)
  o_ref[...] = _dot(t, w2_ref[...]) + b2_ref[...]


def _decode(h, w1, b1, w2p, b2p, block):
  full = lambda i: (0, 0)
  return pl.pallas_call(
      _dec_body,
      grid=(_N // block,),
      in_specs=[
          pl.BlockSpec((block, _H), lambda i: (i, 0)),
          pl.BlockSpec((_H, _H), full),
          pl.BlockSpec((1, _H), full),
          pl.BlockSpec((_H, _H), full),
          pl.BlockSpec((1, _H), full),
      ],
      out_specs=pl.BlockSpec((block, _H), lambda i: (i, 0)),
      out_shape=jax.ShapeDtypeStruct((_N, _H), _F32),
  )(h, w1, b1.reshape(1, -1), w2p, b2p)


def _build_gather_spmem(base, count):
  per_w = count // _NS
  nfull = per_w // _CHUNK
  rem = per_w - nfull * _CHUNK
  npair = nfull // 2
  odd = nfull - 2 * npair
  mesh = plsc.VectorSubcoreMesh(core_axis_name="c", subcore_axis_name="s")

  scratch = [
      pltpu.VMEM((per_w,), jnp.int32),
      pltpu.VMEM((_CHUNK, _H), _F32),
      pltpu.VMEM((_CHUNK, _H), _F32),
      pltpu.VMEM_SHARED((_N, _H), _F32),
      pltpu.SemaphoreType.DMA,
      pltpu.SemaphoreType.DMA,
      pltpu.SemaphoreType.DMA,
      pltpu.SemaphoreType.DMA,
  ]
  if rem:
    scratch.append(pltpu.VMEM((rem, _H), _F32))

  @functools.partial(
      pl.kernel,
      mesh=mesh,
      out_type=jax.ShapeDtypeStruct((2, count, _H), _F32),
      scratch_types=scratch,
  )
  def gather(ab_hbm, idx_hbm, out_hbm, idx_all, rows0, rows1, tbl, sg0, sg1,
             so0, so1, *tail_bufs):
    cid = lax.axis_index("c")
    sid = lax.axis_index("s")
    row0 = sid * per_w
    out_c = out_hbm.at[cid]

    @pl.when(sid == 0)
    def _stage_table():
      pltpu.sync_copy(ab_hbm.at[cid], tbl)

    pltpu.sync_copy(idx_hbm.at[pl.ds(cid * _E + base + row0, per_w)], idx_all)
    plsc.subcore_barrier()

    def body(j, carry):
      i0 = 2 * j
      i1 = i0 + 1

      @pl.when(j > 0)
      def _drain_prev_stores():
        pltpu.make_async_copy(rows0, out_c.at[pl.ds(0, _CHUNK)], so0).wait()
        pltpu.make_async_copy(rows1, out_c.at[pl.ds(0, _CHUNK)], so1).wait()

      g0 = pltpu.async_copy(
          tbl.at[idx_all.at[pl.ds(i0 * _CHUNK, _CHUNK)]], rows0, sg0)
      g1 = pltpu.async_copy(
          tbl.at[idx_all.at[pl.ds(i1 * _CHUNK, _CHUNK)]], rows1, sg1)
      g0.wait()
      pltpu.async_copy(rows0, out_c.at[pl.ds(row0 + i0 * _CHUNK, _CHUNK)], so0)
      g1.wait()
      pltpu.async_copy(rows1, out_c.at[pl.ds(row0 + i1 * _CHUNK, _CHUNK)], so1)
      return carry

    lax.fori_loop(0, npair, body, 0)
    pltpu.make_async_copy(rows0, out_c.at[pl.ds(0, _CHUNK)], so0).wait()
    pltpu.make_async_copy(rows1, out_c.at[pl.ds(0, _CHUNK)], so1).wait()

    if odd:
      i = 2 * npair
      pltpu.async_copy(
          tbl.at[idx_all.at[pl.ds(i * _CHUNK, _CHUNK)]], rows0, sg0).wait()
      pltpu.sync_copy(rows0, out_c.at[pl.ds(row0 + i * _CHUNK, _CHUNK)])

    if rem:
      rowsr, = tail_bufs
      off = nfull * _CHUNK
      pltpu.async_copy(
          tbl.at[idx_all.at[pl.ds(off, rem)]], rowsr, sg1).wait()
      pltpu.sync_copy(rowsr, out_c.at[pl.ds(row0 + off, rem)])

  return gather


def _build_scatter_add(count):
  per_w = count // _NW
  nfull = per_w // _CHUNK
  rem = per_w - nfull * _CHUNK
  npair = nfull // 2
  odd = nfull - 2 * npair
  mesh = plsc.VectorSubcoreMesh(core_axis_name="c", subcore_axis_name="s")

  scratch = [
      pltpu.VMEM((_CHUNK,), jnp.int32),
      pltpu.VMEM((_CHUNK,), jnp.int32),
      pltpu.VMEM((_CHUNK, _H), _F32),
      pltpu.VMEM((_CHUNK, _H), _F32),
      pltpu.VMEM_SHARED((_N, _H), _F32),
      pltpu.SemaphoreType.DMA,
      pltpu.SemaphoreType.DMA,
      pltpu.SemaphoreType.DMA,
      pltpu.SemaphoreType.DMA,
      pltpu.SemaphoreType.DMA,
      pltpu.SemaphoreType.DMA,
  ]
  if rem:
    scratch += [pltpu.VMEM((rem,), jnp.int32), pltpu.VMEM((rem, _H), _F32)]

  @functools.partial(
      pl.kernel,
      mesh=mesh,
      out_type=jax.ShapeDtypeStruct((_NC, _N, _H), _F32),
      scratch_types=scratch,
  )
  def scatter(vals_hbm, idx_hbm, zeros_hbm, out_hbm, idx0, idx1, rows0, rows1,
              acc, si0, si1, sv0, sv1, sa0, sa1, *tail_bufs):
    cid = lax.axis_index("c")
    sid = lax.axis_index("s")
    wid = cid * _NS + sid
    base = wid * per_w

    @pl.when(sid == 0)
    def _zero():
      pltpu.sync_copy(zeros_hbm, acc)

    plsc.subcore_barrier()

    def body(j, carry):
      off0 = base + (2 * j) * _CHUNK
      off1 = off0 + _CHUNK

      @pl.when(j > 0)
      def _drain_prev_adds():
        pltpu.make_async_copy(rows0, acc.at[idx0], sa0).wait()
        pltpu.make_async_copy(rows1, acc.at[idx1], sa1).wait()

      a0 = pltpu.async_copy(idx_hbm.at[pl.ds(off0, _CHUNK)], idx0, si0)
      v0 = pltpu.async_copy(vals_hbm.at[pl.ds(off0, _CHUNK)], rows0, sv0)
      a1 = pltpu.async_copy(idx_hbm.at[pl.ds(off1, _CHUNK)], idx1, si1)
      v1 = pltpu.async_copy(vals_hbm.at[pl.ds(off1, _CHUNK)], rows1, sv1)
      a0.wait()
      v0.wait()
      pltpu.async_copy(rows0, acc.at[idx0], sa0, add=True)
      a1.wait()
      v1.wait()
      pltpu.async_copy(rows1, acc.at[idx1], sa1, add=True)
      return carry

    lax.fori_loop(0, npair, body, 0)
    pltpu.make_async_copy(rows0, acc.at[idx0], sa0).wait()
    pltpu.make_async_copy(rows1, acc.at[idx1], sa1).wait()

    if odd:
      off = base + 2 * npair * _CHUNK
      pltpu.sync_copy(idx_hbm.at[pl.ds(off, _CHUNK)], idx0)
      pltpu.sync_copy(vals_hbm.at[pl.ds(off, _CHUNK)], rows0)
      pltpu.sync_copy(rows0, acc.at[idx0], add=True)

    if rem:
      idxr, rowsr = tail_bufs
      off = base + nfull * _CHUNK
      pltpu.sync_copy(idx_hbm.at[pl.ds(off, rem)], idxr)
      pltpu.sync_copy(vals_hbm.at[pl.ds(off, rem)], rowsr)
      pltpu.sync_copy(rowsr, acc.at[idxr], add=True)

    plsc.subcore_barrier()

    @pl.when(sid == 0)
    def _emit():
      pltpu.sync_copy(acc, out_hbm.at[cid])

  return scatter


def kernel(x, edge_index, edge_attr, enc_n_W1, enc_n_b1, enc_n_W2, enc_n_b2,
           enc_n_g, enc_n_be, enc_e_W1, enc_e_b1, enc_e_W2, enc_e_b2, enc_e_g,
           enc_e_be, pe_W1, pe_b1, pe_W2, pe_b2, pe_g, pe_be, pn_W1, pn_b1,
           pn_W2, pn_b2, pn_g, pn_be, dec_W1, dec_b1, dec_W2, dec_b2):
  src = edge_index[0].astype(jnp.int32)
  dst = edge_index[1].astype(jnp.int32)

  half = _E // 2
  h = _mlp_ln(x, enc_n_W1, enc_n_b1, enc_n_W2, enc_n_b2, enc_n_g, enc_n_be,
              block=1000)
  e_halves = [
      _mlp_ln(edge_attr[:half], enc_e_W1, enc_e_b1, enc_e_W2, enc_e_b2,
              enc_e_g, enc_e_be, block=2000),
      _mlp_ln(edge_attr[half:], enc_e_W1, enc_e_b1, enc_e_W2, enc_e_b2,
              enc_e_g, enc_e_be, block=2000),
  ]

  idx2 = jnp.concatenate([src, dst])
  dst_halves = [dst[:half], dst[half:]]
  zeros = jnp.zeros((_N, _H), _F32)

  gathers = [_build_gather_spmem(0, half), _build_gather_spmem(half, half)]
  scatter = _build_scatter_add(half)

  for s in range(pe_W1.shape[0]):
    w1 = pe_W1[s]
    ab = _ab_tables(h, w1[_H:2 * _H], w1[2 * _H:], block=1000)
    g_halves = [gathers[0](ab, idx2), gathers[1](ab, idx2)]
    parts = []
    for k in (0, 1):
      e_halves[k] = _edge_step(e_halves[k], g_halves[k], w1[:_H], pe_b1[s],
                               pe_W2[s], pe_b2[s], pe_g[s], pe_be[s],
                               block=2000)
      parts.append(scatter(e_halves[k], dst_halves[k], zeros))
    h = _node_step(h, parts[0], parts[1], pn_W1[s][:_H], pn_W1[s][_H:],
                   pn_b1[s], pn_W2[s], pn_b2[s], pn_g[s], pn_be[s], block=1000)

  out = _decode(h, dec_W1, dec_b1, jnp.pad(dec_W2, ((0, 0), (0, _H - 3))),
                jnp.pad(dec_b2, (0, _H - 3)).reshape(1, -1), block=1000)
  return out[:, :3]


# speedup vs baseline: 4.8415x; 1.0179x over previous
import functools

import jax
import jax.numpy as jnp
from jax import lax
from jax.experimental import pallas as pl
from jax.experimental.pallas import tpu as pltpu
from jax.experimental.pallas import tpu_sc as plsc

_N = 10000
_E = 320000
_H = 128
_NC = 2
_NS = 16
_NW = _NC * _NS
_CHUNK = 128

_F32 = jnp.float32


def _ln(u, g, b):
  mu = jnp.mean(u, axis=-1, keepdims=True)
  d = u - mu
  var = jnp.mean(d * d, axis=-1, keepdims=True)
  return d / jnp.sqrt(var + 1e-5) * g + b


def _dot(a, b):
  return jnp.dot(a, b, preferred_element_type=_F32)


def _mlp_ln_body(x_ref, w1_ref, b1_ref, w2_ref, b2_ref, g_ref, be_ref, o_ref):
  t = jnp.maximum(_dot(x_ref[...], w1_ref[...]) + b1_ref[...], 0.0)
  u = _dot(t, w2_ref[...]) + b2_ref[...]
  o_ref[...] = _ln(u, g_ref[...], be_ref[...])


def _mlp_ln(x, w1, b1, w2, b2, g, be, block):
  n, fin = x.shape
  h = w1.shape[1]
  full = lambda i: (0, 0)
  return pl.pallas_call(
      _mlp_ln_body,
      grid=(n // block,),
      in_specs=[
          pl.BlockSpec((block, fin), lambda i: (i, 0)),
          pl.BlockSpec((fin, h), full),
          pl.BlockSpec((1, h), full),
          pl.BlockSpec((h, h), full),
          pl.BlockSpec((1, h), full),
          pl.BlockSpec((1, h), full),
          pl.BlockSpec((1, h), full),
      ],
      out_specs=pl.BlockSpec((block, h), lambda i: (i, 0)),
      out_shape=jax.ShapeDtypeStruct((n, h), _F32),
  )(x, w1, b1.reshape(1, -1), w2, b2.reshape(1, -1), g.reshape(1, -1),
    be.reshape(1, -1))


def _ab_body(h_ref, ws_ref, wd_ref, o_ref):
  hh = h_ref[...]
  o_ref[0] = _dot(hh, ws_ref[...])
  o_ref[1] = _dot(hh, wd_ref[...])


def _ab_tables(h, ws, wd, block):
  full = lambda i: (0, 0)
  return pl.pallas_call(
      _ab_body,
      grid=(_N // block,),
      in_specs=[
          pl.BlockSpec((block, _H), lambda i: (i, 0)),
          pl.BlockSpec((_H, _H), full),
          pl.BlockSpec((_H, _H), full),
      ],
      out_specs=pl.BlockSpec((2, block, _H), lambda i: (0, i, 0)),
      out_shape=jax.ShapeDtypeStruct((2, _N, _H), _F32),
  )(h, ws, wd)


def _edge_body(e_ref, ga_ref, gb_ref, w1_ref, b1_ref, w2_ref, b2_ref, g_ref,
               be_ref, o_ref):
  e = e_ref[...]
  pre = _dot(e, w1_ref[...]) + ga_ref[0] + gb_ref[0] + b1_ref[...]
  t = jnp.maximum(pre, 0.0)
  u = _dot(t, w2_ref[...]) + b2_ref[...]
  o_ref[...] = e + _ln(u, g_ref[...], be_ref[...])


def _edge_step(e, gathered, w1e, b1, w2, b2, g, be, block):
  full = lambda i: (0, 0)
  count = e.shape[0]
  return pl.pallas_call(
      _edge_body,
      grid=(count // block,),
      in_specs=[
          pl.BlockSpec((block, _H), lambda i: (i, 0)),
          pl.BlockSpec((1, block, _H), lambda i: (0, i, 0)),
          pl.BlockSpec((1, block, _H), lambda i: (1, i, 0)),
          pl.BlockSpec((_H, _H), full),
          pl.BlockSpec((1, _H), full),
          pl.BlockSpec((_H, _H), full),
          pl.BlockSpec((1, _H), full),
          pl.BlockSpec((1, _H), full),
          pl.BlockSpec((1, _H), full),
      ],
      out_specs=pl.BlockSpec((block, _H), lambda i: (i, 0)),
      out_shape=jax.ShapeDtypeStruct((count, _H), _F32),
  )(e, gathered, gathered, w1e, b1.reshape(1, -1), w2, b2.reshape(1, -1),
    g.reshape(1, -1), be.reshape(1, -1))


def _node_body(h_ref, p_ref, q_ref, w1h_ref, w1a_ref, b1_ref, w2_ref, b2_ref,
               g_ref, be_ref, o_ref):
  hh = h_ref[...]
  agg = (p_ref[0] + p_ref[1]) + (q_ref[0] + q_ref[1])
  pre = _dot(hh, w1h_ref[...]) + _dot(agg, w1a_ref[...]) + b1_ref[...]
  t = jnp.maximum(pre, 0.0)
  u = _dot(t, w2_ref[...]) + b2_ref[...]
  o_ref[...] = hh + _ln(u, g_ref[...], be_ref[...])


def _node_step(h, parts0, parts1, w1h, w1a, b1, w2, b2, g, be, block):
  full = lambda i: (0, 0)
  return pl.pallas_call(
      _node_body,
      grid=(_N // block,),
      in_specs=[
          pl.BlockSpec((block, _H), lambda i: (i, 0)),
          pl.BlockSpec((2, block, _H), lambda i: (0, i, 0)),
          pl.BlockSpec((2, block, _H), lambda i: (0, i, 0)),
          pl.BlockSpec((_H, _H), full),
          pl.BlockSpec((_H, _H), full),
          pl.BlockSpec((1, _H), full),
          pl.BlockSpec((_H, _H), full),
          pl.BlockSpec((1, _H), full),
          pl.BlockSpec((1, _H), full),
          pl.BlockSpec((1, _H), full),
      ],
      out_specs=pl.BlockSpec((block, _H), lambda i: (i, 0)),
      out_shape=jax.ShapeDtypeStruct((_N, _H), _F32),
  )(h, parts0, parts1, w1h, w1a, b1.reshape(1, -1), w2, b2.reshape(1, -1),
    g.reshape(1, -1), be.reshape(1, -1))


def _dec_body(h_ref, w1_ref, b1_ref, w2_ref, b2_ref, o_ref):
  t = jnp.maximum(_dot(h_ref[...], w1_ref[...]) + b1_ref[...], 0.0)
  o_ref[...] = _dot(t, w2_ref[...]) + b2_ref[...]


def _decode(h, w1, b1, w2p, b2p, block):
  full = lambda i: (0, 0)
  return pl.pallas_call(
      _dec_body,
      grid=(_N // block,),
      in_specs=[
          pl.BlockSpec((block, _H), lambda i: (i, 0)),
          pl.BlockSpec((_H, _H), full),
          pl.BlockSpec((1, _H), full),
          pl.BlockSpec((_H, _H), full),
          pl.BlockSpec((1, _H), full),
      ],
      out_specs=pl.BlockSpec((block, _H), lambda i: (i, 0)),
      out_shape=jax.ShapeDtypeStruct((_N, _H), _F32),
  )(h, w1, b1.reshape(1, -1), w2p, b2p)


def _build_gather_spmem(base, count):
  per_w = count // _NS
  nfull = per_w // _CHUNK
  rem = per_w - nfull * _CHUNK
  npair = nfull // 2
  odd = nfull - 2 * npair
  mesh = plsc.VectorSubcoreMesh(core_axis_name="c", subcore_axis_name="s")

  scratch = [
      pltpu.VMEM((per_w,), jnp.int32),
      pltpu.VMEM((_CHUNK, _H), _F32),
      pltpu.VMEM((_CHUNK, _H), _F32),
      pltpu.VMEM_SHARED((_N, _H), _F32),
      pltpu.SemaphoreType.DMA,
      pltpu.SemaphoreType.DMA,
      pltpu.SemaphoreType.DMA,
      pltpu.SemaphoreType.DMA,
  ]
  if rem:
    scratch.append(pltpu.VMEM((rem, _H), _F32))

  @functools.partial(
      pl.kernel,
      mesh=mesh,
      out_type=jax.ShapeDtypeStruct((2, count, _H), _F32),
      scratch_types=scratch,
  )
  def gather(ab_hbm, idx_hbm, out_hbm, idx_all, rows0, rows1, tbl, sg0, sg1,
             so0, so1, *tail_bufs):
    cid = lax.axis_index("c")
    sid = lax.axis_index("s")
    row0 = sid * per_w
    out_c = out_hbm.at[cid]

    @pl.when(sid == 0)
    def _stage_table():
      pltpu.sync_copy(ab_hbm.at[cid], tbl)

    pltpu.sync_copy(idx_hbm.at[pl.ds(cid * _E + base + row0, per_w)], idx_all)
    plsc.subcore_barrier()

    def body(j, carry):
      i0 = 2 * j
      i1 = i0 + 1

      @pl.when(j > 0)
      def _drain0():
        pltpu.make_async_copy(rows0, out_c.at[pl.ds(0, _CHUNK)], so0).wait()

      g0 = pltpu.async_copy(
          tbl.at[idx_all.at[pl.ds(i0 * _CHUNK, _CHUNK)]], rows0, sg0)

      @pl.when(j > 0)
      def _drain1():
        pltpu.make_async_copy(rows1, out_c.at[pl.ds(0, _CHUNK)], so1).wait()

      g1 = pltpu.async_copy(
          tbl.at[idx_all.at[pl.ds(i1 * _CHUNK, _CHUNK)]], rows1, sg1)
      g0.wait()
      pltpu.async_copy(rows0, out_c.at[pl.ds(row0 + i0 * _CHUNK, _CHUNK)], so0)
      g1.wait()
      pltpu.async_copy(rows1, out_c.at[pl.ds(row0 + i1 * _CHUNK, _CHUNK)], so1)
      return carry

    lax.fori_loop(0, npair, body, 0)
    pltpu.make_async_copy(rows0, out_c.at[pl.ds(0, _CHUNK)], so0).wait()
    pltpu.make_async_copy(rows1, out_c.at[pl.ds(0, _CHUNK)], so1).wait()

    if odd:
      i = 2 * npair
      pltpu.async_copy(
          tbl.at[idx_all.at[pl.ds(i * _CHUNK, _CHUNK)]], rows0, sg0).wait()
      pltpu.sync_copy(rows0, out_c.at[pl.ds(row0 + i * _CHUNK, _CHUNK)])

    if rem:
      rowsr, = tail_bufs
      off = nfull * _CHUNK
      pltpu.async_copy(
          tbl.at[idx_all.at[pl.ds(off, rem)]], rowsr, sg1).wait()
      pltpu.sync_copy(rowsr, out_c.at[pl.ds(row0 + off, rem)])

  return gather


def _build_scatter_add(count):
  per_w = count // _NW
  nfull = per_w // _CHUNK
  rem = per_w - nfull * _CHUNK
  ntrip = nfull // 3
  left = nfull - 3 * ntrip
  mesh = plsc.VectorSubcoreMesh(core_axis_name="c", subcore_axis_name="s")

  scratch = [
      pltpu.VMEM((_CHUNK,), jnp.int32),
      pltpu.VMEM((_CHUNK,), jnp.int32),
      pltpu.VMEM((_CHUNK,), jnp.int32),
      pltpu.VMEM((_CHUNK, _H), _F32),
      pltpu.VMEM((_CHUNK, _H), _F32),
      pltpu.VMEM((_CHUNK, _H), _F32),
      pltpu.VMEM_SHARED((_N, _H), _F32),
  ] + [pltpu.SemaphoreType.DMA] * 9
  if rem:
    scratch += [pltpu.VMEM((rem,), jnp.int32), pltpu.VMEM((rem, _H), _F32)]

  @functools.partial(
      pl.kernel,
      mesh=mesh,
      out_type=jax.ShapeDtypeStruct((_NC, _N, _H), _F32),
      scratch_types=scratch,
  )
  def scatter(vals_hbm, idx_hbm, zeros_hbm, out_hbm, idx0, idx1, idx2, rows0,
              rows1, rows2, acc, si0, si1, si2, sv0, sv1, sv2, sa0, sa1, sa2,
              *tail_bufs):
    cid = lax.axis_index("c")
    sid = lax.axis_index("s")
    wid = cid * _NS + sid
    base = wid * per_w
    idx = (idx0, idx1, idx2)
    rows = (rows0, rows1, rows2)
    si = (si0, si1, si2)
    sv = (sv0, sv1, sv2)
    sa = (sa0, sa1, sa2)

    @pl.when(sid == 0)
    def _zero():
      pltpu.sync_copy(zeros_hbm, acc)

    plsc.subcore_barrier()

    def body(j, carry):
      i0 = 3 * j
      pend = []
      for b in range(3):
        off = base + (i0 + b) * _CHUNK

        @pl.when(j > 0)
        def _drain(b=b):
          pltpu.make_async_copy(rows[b], acc.at[idx[b]], sa[b]).wait()

        a = pltpu.async_copy(idx_hbm.at[pl.ds(off, _CHUNK)], idx[b], si[b])
        v = pltpu.async_copy(vals_hbm.at[pl.ds(off, _CHUNK)], rows[b], sv[b])
        pend.append((a, v))
      for b in range(3):
        a, v = pend[b]
        a.wait()
        v.wait()
        pltpu.async_copy(rows[b], acc.at[idx[b]], sa[b], add=True)
      return carry

    lax.fori_loop(0, ntrip, body, 0)
    for b in range(3):
      pltpu.make_async_copy(rows[b], acc.at[idx[b]], sa[b]).wait()

    for t in range(left):
      off = base + (3 * ntrip + t) * _CHUNK
      pltpu.sync_copy(idx_hbm.at[pl.ds(off, _CHUNK)], idx0)
      pltpu.sync_copy(vals_hbm.at[pl.ds(off, _CHUNK)], rows0)
      pltpu.sync_copy(rows0, acc.at[idx0], add=True)

    if rem:
      idxr, rowsr = tail_bufs
      off = base + nfull * _CHUNK
      pltpu.sync_copy(idx_hbm.at[pl.ds(off, rem)], idxr)
      pltpu.sync_copy(vals_hbm.at[pl.ds(off, rem)], rowsr)
      pltpu.sync_copy(rowsr, acc.at[idxr], add=True)

    plsc.subcore_barrier()

    @pl.when(sid == 0)
    def _emit():
      pltpu.sync_copy(acc, out_hbm.at[cid])

  return scatter


def kernel(x, edge_index, edge_attr, enc_n_W1, enc_n_b1, enc_n_W2, enc_n_b2,
           enc_n_g, enc_n_be, enc_e_W1, enc_e_b1, enc_e_W2, enc_e_b2, enc_e_g,
           enc_e_be, pe_W1, pe_b1, pe_W2, pe_b2, pe_g, pe_be, pn_W1, pn_b1,
           pn_W2, pn_b2, pn_g, pn_be, dec_W1, dec_b1, dec_W2, dec_b2):
  src = edge_index[0].astype(jnp.int32)
  dst = edge_index[1].astype(jnp.int32)

  half = _E // 2
  h = _mlp_ln(x, enc_n_W1, enc_n_b1, enc_n_W2, enc_n_b2, enc_n_g, enc_n_be,
              block=1000)
  e_halves = [
      _mlp_ln(edge_attr[:half], enc_e_W1, enc_e_b1, enc_e_W2, enc_e_b2,
              enc_e_g, enc_e_be, block=2000),
      _mlp_ln(edge_attr[half:], enc_e_W1, enc_e_b1, enc_e_W2, enc_e_b2,
              enc_e_g, enc_e_be, block=2000),
  ]

  idx2 = jnp.concatenate([src, dst])
  dst_halves = [dst[:half], dst[half:]]
  zeros = jnp.zeros((_N, _H), _F32)

  gathers = [_build_gather_spmem(0, half), _build_gather_spmem(half, half)]
  scatter = _build_scatter_add(half)

  for s in range(pe_W1.shape[0]):
    w1 = pe_W1[s]
    ab = _ab_tables(h, w1[_H:2 * _H], w1[2 * _H:], block=1000)
    g_halves = [gathers[0](ab, idx2), gathers[1](ab, idx2)]
    parts = []
    for k in (0, 1):
      e_halves[k] = _edge_step(e_halves[k], g_halves[k], w1[:_H], pe_b1[s],
                               pe_W2[s], pe_b2[s], pe_g[s], pe_be[s],
                               block=2000)
      parts.append(scatter(e_halves[k], dst_halves[k], zeros))
    h = _node_step(h, parts[0], parts[1], pn_W1[s][:_H], pn_W1[s][_H:],
                   pn_b1[s], pn_W2[s], pn_b2[s], pn_g[s], pn_be[s], block=1000)

  out = _decode(h, dec_W1, dec_b1, jnp.pad(dec_W2, ((0, 0), (0, _H - 3))),
                jnp.pad(dec_b2, (0, _H - 3)).reshape(1, -1), block=1000)
  return out[:, :3]


# speedup vs baseline: 4.9030x; 1.0127x over previous
import functools

import jax
import jax.numpy as jnp
from jax import lax
from jax.experimental import pallas as pl
from jax.experimental.pallas import tpu as pltpu
from jax.experimental.pallas import tpu_sc as plsc

_N = 10000
_E = 320000
_H = 128
_NC = 2
_NS = 16
_NW = _NC * _NS
_CHUNK = 128

_F32 = jnp.float32


def _ln(u, g, b):
  mu = jnp.mean(u, axis=-1, keepdims=True)
  d = u - mu
  var = jnp.mean(d * d, axis=-1, keepdims=True)
  return d / jnp.sqrt(var + 1e-5) * g + b


def _dot(a, b):
  return jnp.dot(a, b, preferred_element_type=_F32)


def _mlp_ln_body(x_ref, w1_ref, b1_ref, w2_ref, b2_ref, g_ref, be_ref, o_ref):
  t = jnp.maximum(_dot(x_ref[...], w1_ref[...]) + b1_ref[...], 0.0)
  u = _dot(t, w2_ref[...]) + b2_ref[...]
  o_ref[...] = _ln(u, g_ref[...], be_ref[...])


def _mlp_ln(x, w1, b1, w2, b2, g, be, block):
  n, fin = x.shape
  h = w1.shape[1]
  full = lambda i: (0, 0)
  return pl.pallas_call(
      _mlp_ln_body,
      grid=(n // block,),
      in_specs=[
          pl.BlockSpec((block, fin), lambda i: (i, 0)),
          pl.BlockSpec((fin, h), full),
          pl.BlockSpec((1, h), full),
          pl.BlockSpec((h, h), full),
          pl.BlockSpec((1, h), full),
          pl.BlockSpec((1, h), full),
          pl.BlockSpec((1, h), full),
      ],
      out_specs=pl.BlockSpec((block, h), lambda i: (i, 0)),
      out_shape=jax.ShapeDtypeStruct((n, h), _F32),
  )(x, w1, b1.reshape(1, -1), w2, b2.reshape(1, -1), g.reshape(1, -1),
    be.reshape(1, -1))


def _ab_body(h_ref, ws_ref, wd_ref, o_ref):
  hh = h_ref[...]
  o_ref[0] = _dot(hh, ws_ref[...])
  o_ref[1] = _dot(hh, wd_ref[...])


def _ab_tables(h, ws, wd, block):
  full = lambda i: (0, 0)
  return pl.pallas_call(
      _ab_body,
      grid=(_N // block,),
      in_specs=[
          pl.BlockSpec((block, _H), lambda i: (i, 0)),
          pl.BlockSpec((_H, _H), full),
          pl.BlockSpec((_H, _H), full),
      ],
      out_specs=pl.BlockSpec((2, block, _H), lambda i: (0, i, 0)),
      out_shape=jax.ShapeDtypeStruct((2, _N, _H), _F32),
  )(h, ws, wd)


def _edge_body(e_ref, ga_ref, gb_ref, w1_ref, b1_ref, w2_ref, b2_ref, g_ref,
               be_ref, o_ref):
  e = e_ref[...]
  pre = _dot(e, w1_ref[...]) + ga_ref[0] + gb_ref[0] + b1_ref[...]
  t = jnp.maximum(pre, 0.0)
  u = _dot(t, w2_ref[...]) + b2_ref[...]
  o_ref[...] = e + _ln(u, g_ref[...], be_ref[...])


def _edge_step(e, gathered, w1e, b1, w2, b2, g, be, block):
  full = lambda i: (0, 0)
  count = e.shape[0]
  return pl.pallas_call(
      _edge_body,
      grid=(count // block,),
      in_specs=[
          pl.BlockSpec((block, _H), lambda i: (i, 0)),
          pl.BlockSpec((1, block, _H), lambda i: (0, i, 0)),
          pl.BlockSpec((1, block, _H), lambda i: (1, i, 0)),
          pl.BlockSpec((_H, _H), full),
          pl.BlockSpec((1, _H), full),
          pl.BlockSpec((_H, _H), full),
          pl.BlockSpec((1, _H), full),
          pl.BlockSpec((1, _H), full),
          pl.BlockSpec((1, _H), full),
      ],
      out_specs=pl.BlockSpec((block, _H), lambda i: (i, 0)),
      out_shape=jax.ShapeDtypeStruct((count, _H), _F32),
  )(e, gathered, gathered, w1e, b1.reshape(1, -1), w2, b2.reshape(1, -1),
    g.reshape(1, -1), be.reshape(1, -1))


def _node_body(h_ref, p_ref, q_ref, w1h_ref, w1a_ref, b1_ref, w2_ref, b2_ref,
               g_ref, be_ref, o_ref):
  hh = h_ref[...]
  agg = (p_ref[0] + p_ref[1]) + (q_ref[0] + q_ref[1])
  pre = _dot(hh, w1h_ref[...]) + _dot(agg, w1a_ref[...]) + b1_ref[...]
  t = jnp.maximum(pre, 0.0)
  u = _dot(t, w2_ref[...]) + b2_ref[...]
  o_ref[...] = hh + _ln(u, g_ref[...], be_ref[...])


def _node_body_ab(h_ref, p_ref, q_ref, w1h_ref, w1a_ref, b1_ref, w2_ref,
                  b2_ref, g_ref, be_ref, ws_ref, wd_ref, o_ref, ab_ref):
  hh = h_ref[...]
  agg = (p_ref[0] + p_ref[1]) + (q_ref[0] + q_ref[1])
  pre = _dot(hh, w1h_ref[...]) + _dot(agg, w1a_ref[...]) + b1_ref[...]
  t = jnp.maximum(pre, 0.0)
  u = _dot(t, w2_ref[...]) + b2_ref[...]
  hn = hh + _ln(u, g_ref[...], be_ref[...])
  o_ref[...] = hn
  ab_ref[0] = _dot(hn, ws_ref[...])
  ab_ref[1] = _dot(hn, wd_ref[...])


def _node_step(h, parts0, parts1, w1h, w1a, b1, w2, b2, g, be, block,
               ws_next=None, wd_next=None):
  full = lambda i: (0, 0)
  in_specs = [
      pl.BlockSpec((block, _H), lambda i: (i, 0)),
      pl.BlockSpec((2, block, _H), lambda i: (0, i, 0)),
      pl.BlockSpec((2, block, _H), lambda i: (0, i, 0)),
      pl.BlockSpec((_H, _H), full),
      pl.BlockSpec((_H, _H), full),
      pl.BlockSpec((1, _H), full),
      pl.BlockSpec((_H, _H), full),
      pl.BlockSpec((1, _H), full),
      pl.BlockSpec((1, _H), full),
      pl.BlockSpec((1, _H), full),
  ]
  args = [h, parts0, parts1, w1h, w1a, b1.reshape(1, -1), w2,
          b2.reshape(1, -1), g.reshape(1, -1), be.reshape(1, -1)]
  if ws_next is None:
    return pl.pallas_call(
        _node_body,
        grid=(_N // block,),
        in_specs=in_specs,
        out_specs=pl.BlockSpec((block, _H), lambda i: (i, 0)),
        out_shape=jax.ShapeDtypeStruct((_N, _H), _F32),
    )(*args)
  return pl.pallas_call(
      _node_body_ab,
      grid=(_N // block,),
      in_specs=in_specs + [pl.BlockSpec((_H, _H), full),
                           pl.BlockSpec((_H, _H), full)],
      out_specs=[pl.BlockSpec((block, _H), lambda i: (i, 0)),
                 pl.BlockSpec((2, block, _H), lambda i: (0, i, 0))],
      out_shape=[jax.ShapeDtypeStruct((_N, _H), _F32),
                 jax.ShapeDtypeStruct((2, _N, _H), _F32)],
  )(*args, ws_next, wd_next)


def _dec_body(h_ref, w1_ref, b1_ref, w2_ref, b2_ref, o_ref):
  t = jnp.maximum(_dot(h_ref[...], w1_ref[...]) + b1_ref[...], 0.0)
  o_ref[...] = _dot(t, w2_ref[...]) + b2_ref[...]


def _decode(h, w1, b1, w2p, b2p, block):
  full = lambda i: (0, 0)
  return pl.pallas_call(
      _dec_body,
      grid=(_N // block,),
      in_specs=[
          pl.BlockSpec((block, _H), lambda i: (i, 0)),
          pl.BlockSpec((_H, _H), full),
          pl.BlockSpec((1, _H), full),
          pl.BlockSpec((_H, _H), full),
          pl.BlockSpec((1, _H), full),
      ],
      out_specs=pl.BlockSpec((block, _H), lambda i: (i, 0)),
      out_shape=jax.ShapeDtypeStruct((_N, _H), _F32),
  )(h, w1, b1.reshape(1, -1), w2p, b2p)


def _build_gather_spmem(base, count):
  per_w = count // _NS
  nfull = per_w // _CHUNK
  rem = per_w - nfull * _CHUNK
  npair = nfull // 2
  odd = nfull - 2 * npair
  mesh = plsc.VectorSubcoreMesh(core_axis_name="c", subcore_axis_name="s")

  scratch = [
      pltpu.VMEM((per_w,), jnp.int32),
      pltpu.VMEM((_CHUNK, _H), _F32),
      pltpu.VMEM((_CHUNK, _H), _F32),
      pltpu.VMEM_SHARED((_N, _H), _F32),
      pltpu.SemaphoreType.DMA,
      pltpu.SemaphoreType.DMA,
      pltpu.SemaphoreType.DMA,
      pltpu.SemaphoreType.DMA,
  ]
  if rem:
    scratch.append(pltpu.VMEM((rem, _H), _F32))

  @functools.partial(
      pl.kernel,
      mesh=mesh,
      out_type=jax.ShapeDtypeStruct((2, count, _H), _F32),
      scratch_types=scratch,
  )
  def gather(ab_hbm, idx_hbm, out_hbm, idx_all, rows0, rows1, tbl, sg0, sg1,
             so0, so1, *tail_bufs):
    cid = lax.axis_index("c")
    sid = lax.axis_index("s")
    row0 = sid * per_w
    out_c = out_hbm.at[cid]

    @pl.when(sid == 0)
    def _stage_table():
      pltpu.sync_copy(ab_hbm.at[cid], tbl)

    pltpu.sync_copy(idx_hbm.at[pl.ds(cid * _E + base + row0, per_w)], idx_all)
    plsc.subcore_barrier()

    def body(j, carry):
      i0 = 2 * j
      i1 = i0 + 1

      @pl.when(j > 0)
      def _drain0():
        pltpu.make_async_copy(rows0, out_c.at[pl.ds(0, _CHUNK)], so0).wait()

      g0 = pltpu.async_copy(
          tbl.at[idx_all.at[pl.ds(i0 * _CHUNK, _CHUNK)]], rows0, sg0)

      @pl.when(j > 0)
      def _drain1():
        pltpu.make_async_copy(rows1, out_c.at[pl.ds(0, _CHUNK)], so1).wait()

      g1 = pltpu.async_copy(
          tbl.at[idx_all.at[pl.ds(i1 * _CHUNK, _CHUNK)]], rows1, sg1)
      g0.wait()
      pltpu.async_copy(rows0, out_c.at[pl.ds(row0 + i0 * _CHUNK, _CHUNK)], so0)
      g1.wait()
      pltpu.async_copy(rows1, out_c.at[pl.ds(row0 + i1 * _CHUNK, _CHUNK)], so1)
      return carry

    lax.fori_loop(0, npair, body, 0)
    pltpu.make_async_copy(rows0, out_c.at[pl.ds(0, _CHUNK)], so0).wait()
    pltpu.make_async_copy(rows1, out_c.at[pl.ds(0, _CHUNK)], so1).wait()

    if odd:
      i = 2 * npair
      pltpu.async_copy(
          tbl.at[idx_all.at[pl.ds(i * _CHUNK, _CHUNK)]], rows0, sg0).wait()
      pltpu.sync_copy(rows0, out_c.at[pl.ds(row0 + i * _CHUNK, _CHUNK)])

    if rem:
      rowsr, = tail_bufs
      off = nfull * _CHUNK
      pltpu.async_copy(
          tbl.at[idx_all.at[pl.ds(off, rem)]], rowsr, sg1).wait()
      pltpu.sync_copy(rowsr, out_c.at[pl.ds(row0 + off, rem)])

  return gather


def _build_scatter_add(count):
  per_w = count // _NW
  nfull = per_w // _CHUNK
  rem = per_w - nfull * _CHUNK
  ntrip = nfull // 3
  left = nfull - 3 * ntrip
  mesh = plsc.VectorSubcoreMesh(core_axis_name="c", subcore_axis_name="s")

  scratch = [
      pltpu.VMEM((_CHUNK,), jnp.int32),
      pltpu.VMEM((_CHUNK,), jnp.int32),
      pltpu.VMEM((_CHUNK,), jnp.int32),
      pltpu.VMEM((_CHUNK, _H), _F32),
      pltpu.VMEM((_CHUNK, _H), _F32),
      pltpu.VMEM((_CHUNK, _H), _F32),
      pltpu.VMEM_SHARED((_N, _H), _F32),
  ] + [pltpu.SemaphoreType.DMA] * 9
  if rem:
    scratch += [pltpu.VMEM((rem,), jnp.int32), pltpu.VMEM((rem, _H), _F32)]

  @functools.partial(
      pl.kernel,
      mesh=mesh,
      out_type=jax.ShapeDtypeStruct((_NC, _N, _H), _F32),
      scratch_types=scratch,
  )
  def scatter(vals_hbm, idx_hbm, zeros_hbm, out_hbm, idx0, idx1, idx2, rows0,
              rows1, rows2, acc, si0, si1, si2, sv0, sv1, sv2, sa0, sa1, sa2,
              *tail_bufs):
    cid = lax.axis_index("c")
    sid = lax.axis_index("s")
    wid = cid * _NS + sid
    base = wid * per_w
    idx = (idx0, idx1, idx2)
    rows = (rows0, rows1, rows2)
    si = (si0, si1, si2)
    sv = (sv0, sv1, sv2)
    sa = (sa0, sa1, sa2)

    @pl.when(sid == 0)
    def _zero():
      pltpu.sync_copy(zeros_hbm, acc)

    plsc.subcore_barrier()

    def body(j, carry):
      i0 = 3 * j
      pend = []
      for b in range(3):
        off = base + (i0 + b) * _CHUNK

        @pl.when(j > 0)
        def _drain(b=b):
          pltpu.make_async_copy(rows[b], acc.at[idx[b]], sa[b]).wait()

        a = pltpu.async_copy(idx_hbm.at[pl.ds(off, _CHUNK)], idx[b], si[b])
        v = pltpu.async_copy(vals_hbm.at[pl.ds(off, _CHUNK)], rows[b], sv[b])
        pend.append((a, v))
      for b in range(3):
        a, v = pend[b]
        a.wait()
        v.wait()
        pltpu.async_copy(rows[b], acc.at[idx[b]], sa[b], add=True)
      return carry

    lax.fori_loop(0, ntrip, body, 0)
    for b in range(3):
      pltpu.make_async_copy(rows[b], acc.at[idx[b]], sa[b]).wait()

    for t in range(left):
      off = base + (3 * ntrip + t) * _CHUNK
      pltpu.sync_copy(idx_hbm.at[pl.ds(off, _CHUNK)], idx0)
      pltpu.sync_copy(vals_hbm.at[pl.ds(off, _CHUNK)], rows0)
      pltpu.sync_copy(rows0, acc.at[idx0], add=True)

    if rem:
      idxr, rowsr = tail_bufs
      off = base + nfull * _CHUNK
      pltpu.sync_copy(idx_hbm.at[pl.ds(off, rem)], idxr)
      pltpu.sync_copy(vals_hbm.at[pl.ds(off, rem)], rowsr)
      pltpu.sync_copy(rowsr, acc.at[idxr], add=True)

    plsc.subcore_barrier()

    @pl.when(sid == 0)
    def _emit():
      pltpu.sync_copy(acc, out_hbm.at[cid])

  return scatter


def kernel(x, edge_index, edge_attr, enc_n_W1, enc_n_b1, enc_n_W2, enc_n_b2,
           enc_n_g, enc_n_be, enc_e_W1, enc_e_b1, enc_e_W2, enc_e_b2, enc_e_g,
           enc_e_be, pe_W1, pe_b1, pe_W2, pe_b2, pe_g, pe_be, pn_W1, pn_b1,
           pn_W2, pn_b2, pn_g, pn_be, dec_W1, dec_b1, dec_W2, dec_b2):
  src = edge_index[0].astype(jnp.int32)
  dst = edge_index[1].astype(jnp.int32)

  half = _E // 2
  h = _mlp_ln(x, enc_n_W1, enc_n_b1, enc_n_W2, enc_n_b2, enc_n_g, enc_n_be,
              block=1000)
  e_halves = [
      _mlp_ln(edge_attr[:half], enc_e_W1, enc_e_b1, enc_e_W2, enc_e_b2,
              enc_e_g, enc_e_be, block=2000),
      _mlp_ln(edge_attr[half:], enc_e_W1, enc_e_b1, enc_e_W2, enc_e_b2,
              enc_e_g, enc_e_be, block=2000),
  ]

  idx2 = jnp.concatenate([src, dst])
  dst_halves = [dst[:half], dst[half:]]
  zeros = jnp.zeros((_N, _H), _F32)

  gathers = [_build_gather_spmem(0, half), _build_gather_spmem(half, half)]
  scatter = _build_scatter_add(half)

  steps = pe_W1.shape[0]
  ab = _ab_tables(h, pe_W1[0][_H:2 * _H], pe_W1[0][2 * _H:], block=1000)
  for s in range(steps):
    w1 = pe_W1[s]
    g_halves = [gathers[0](ab, idx2), gathers[1](ab, idx2)]
    parts = []
    for k in (0, 1):
      e_halves[k] = _edge_step(e_halves[k], g_halves[k], w1[:_H], pe_b1[s],
                               pe_W2[s], pe_b2[s], pe_g[s], pe_be[s],
                               block=2000)
      parts.append(scatter(e_halves[k], dst_halves[k], zeros))
    if s + 1 < steps:
      h, ab = _node_step(h, parts[0], parts[1], pn_W1[s][:_H], pn_W1[s][_H:],
                         pn_b1[s], pn_W2[s], pn_b2[s], pn_g[s], pn_be[s],
                         block=1000, ws_next=pe_W1[s + 1][_H:2 * _H],
                         wd_next=pe_W1[s + 1][2 * _H:])
    else:
      h = _node_step(h, parts[0], parts[1], pn_W1[s][:_H], pn_W1[s][_H:],
                     pn_b1[s], pn_W2[s], pn_b2[s], pn_g[s], pn_be[s],
                     block=1000)

  out = _decode(h, dec_W1, dec_b1, jnp.pad(dec_W2, ((0, 0), (0, _H - 3))),
                jnp.pad(dec_b2, (0, _H - 3)).reshape(1, -1), block=1000)
  return out[:, :3]
